# baseline (speedup 1.0000x reference)
import functools

import jax
import jax.numpy as jnp
from jax import lax
from jax.experimental import pallas as pl
from jax.experimental.pallas import tpu as pltpu
from jax.experimental.pallas import tpu_sc as plsc

D = 128
N = 20000
E = 320000
NC = 2
NS = 16
NW = NC * NS
K = 128
NB = 80
EW = NB * K
EPAD = NW * EW
ACC = 9600
SENT = 9599
STRIPE = ACC // NS
BR = 400
NBLK = N // BR
AB = ACC // BR


def _sc_segment_sum(table, gidx, sidx, zrows):
    mesh = plsc.VectorSubcoreMesh(
        core_axis_name="c", subcore_axis_name="s",
        num_cores=NC, num_subcores=NS)

    @functools.partial(
        pl.kernel,
        out_type=jax.ShapeDtypeStruct((NC, ACC, D), jnp.float32),
        mesh=mesh,
        scratch_types=[
            pltpu.VMEM((NB, K), jnp.int32),
            pltpu.VMEM((NB, K), jnp.int32),
            pltpu.VMEM((K, D), jnp.float32),
            pltpu.VMEM((K, D), jnp.float32),
            pltpu.VMEM_SHARED((ACC, D), jnp.float32),
            pltpu.SemaphoreType.DMA,
            pltpu.SemaphoreType.DMA,
        ],
    )
    def seg_kernel(table_h, gidx_h, sidx_h, z_h, out_h,
                   gall, sall, rowsA, rowsB, acc, semA, semB):
        cid = lax.axis_index("c")
        sid = lax.axis_index("s")
        wid = sid * NC + cid
        pltpu.sync_copy(gidx_h.at[wid], gall)
        pltpu.sync_copy(sidx_h.at[wid], sall)
        pltpu.async_copy(table_h.at[gall.at[0]], rowsA, semA)
        pltpu.sync_copy(z_h, acc.at[pl.ds(sid * STRIPE, STRIPE)])
        plsc.subcore_barrier()

        def waitA(b):
            pltpu.make_async_copy(table_h.at[gall.at[b]], rowsA, semA).wait()

        def waitB(b):
            pltpu.make_async_copy(table_h.at[gall.at[b]], rowsB, semB).wait()

        def body(t, carry):
            b0 = 2 * t
            b1 = b0 + 1
            pltpu.async_copy(table_h.at[gall.at[b1]], rowsB, semB)
            waitA(b0)
            pltpu.sync_copy(rowsA, acc.at[sall.at[b0]], add=True)
            pltpu.async_copy(table_h.at[gall.at[b0 + 2]], rowsA, semA)
            waitB(b1)
            pltpu.sync_copy(rowsB, acc.at[sall.at[b1]], add=True)
            return carry

        lax.fori_loop(0, NB // 2 - 1, body, 0)
        pltpu.async_copy(table_h.at[gall.at[NB - 1]], rowsB, semB)
        waitA(NB - 2)
        pltpu.sync_copy(rowsA, acc.at[sall.at[NB - 2]], add=True)
        waitB(NB - 1)
        pltpu.sync_copy(rowsB, acc.at[sall.at[NB - 1]], add=True)
        plsc.subcore_barrier()
        pltpu.sync_copy(acc.at[pl.ds(sid * STRIPE, STRIPE)],
                        out_h.at[cid, pl.ds(sid * STRIPE, STRIPE)])

    return seg_kernel(table, gidx, sidx, zrows)


def _mlp_combine(x, W1, b1, W2, b2, cWx, cWh, cb, agg):
    h = jnp.maximum(jnp.dot(agg, W1, preferred_element_type=jnp.float32) + b1, 0.0)
    h = jnp.maximum(jnp.dot(h, W2, preferred_element_type=jnp.float32) + b2, 0.0)
    cand = jnp.dot(x, cWx, preferred_element_type=jnp.float32)
    cand = cand + jnp.dot(h, cWh, preferred_element_type=jnp.float32) + cb
    return jnp.maximum(cand, 0.0)


def _combine_pass1(n0a, xs, accA, accB, W1, b1, W2, b2, cWx, cWh, cb):
    def body(n0s, xsr, aAr, aBr, W1r, b1r, W2r, b2r, cWxr, cWhr, cbr,
             outr, candr):
        i = pl.program_id(0)
        rows = i * BR + lax.broadcasted_iota(jnp.int32, (BR, 1), 0)
        agg = jnp.where(rows < 9500, aAr[...] + aBr[...], 0.0)
        cand = _mlp_combine(xsr[...], W1r[...], b1r[...], W2r[...], b2r[...],
                            cWxr[...], cWhr[...], cbr[...], agg)
        candr[...] = cand
        outr[...] = jnp.where(rows < n0s[0], cand, xsr[...])

    w = lambda i, s: (0, 0)
    grid_spec = pltpu.PrefetchScalarGridSpec(
        num_scalar_prefetch=1,
        grid=(NBLK,),
        in_specs=[
            pl.BlockSpec((BR, D), lambda i, s: (i, 0)),
            pl.BlockSpec((BR, D), lambda i, s: (jnp.minimum(i, AB - 1), 0)),
            pl.BlockSpec((BR, D), lambda i, s: (jnp.minimum(i, AB - 1), 0)),
            pl.BlockSpec((D, D), w), pl.BlockSpec((1, D), w),
            pl.BlockSpec((D, D), w), pl.BlockSpec((1, D), w),
            pl.BlockSpec((D, D), w), pl.BlockSpec((D, D), w),
            pl.BlockSpec((1, D), w),
        ],
        out_specs=[pl.BlockSpec((BR, D), lambda i, s: (i, 0)),
                   pl.BlockSpec((BR, D), lambda i, s: (i, 0))],
    )
    return pl.pallas_call(
        body,
        grid_spec=grid_spec,
        out_shape=[jax.ShapeDtypeStruct((N, D), jnp.float32),
                   jax.ShapeDtypeStruct((N, D), jnp.float32)],
    )(n0a, xs, accA, accB, W1, b1, W2, b2, cWx, cWh, cb)


def _combine_pass2(n0a, xs, agg2, W1, b1, W2, b2, cWx, cWh, cb):
    def body(n0s, xsr, aggr, W1r, b1r, W2r, b2r, cWxr, cWhr, cbr, outr):
        i = pl.program_id(0)
        rows = i * BR + lax.broadcasted_iota(jnp.int32, (BR, 1), 0)
        cand = _mlp_combine(xsr[...], W1r[...], b1r[...], W2r[...], b2r[...],
                            cWxr[...], cWhr[...], cbr[...], aggr[...])
        outr[...] = jnp.where(rows >= n0s[0], cand, xsr[...])

    w = lambda i, s: (0, 0)
    grid_spec = pltpu.PrefetchScalarGridSpec(
        num_scalar_prefetch=1,
        grid=(NBLK,),
        in_specs=[
            pl.BlockSpec((BR, D), lambda i, s: (i, 0)),
            pl.BlockSpec((BR, D), lambda i, s: (i, 0)),
            pl.BlockSpec((D, D), w), pl.BlockSpec((1, D), w),
            pl.BlockSpec((D, D), w), pl.BlockSpec((1, D), w),
            pl.BlockSpec((D, D), w), pl.BlockSpec((D, D), w),
            pl.BlockSpec((1, D), w),
        ],
        out_specs=pl.BlockSpec((BR, D), lambda i, s: (i, 0)),
    )
    return pl.pallas_call(
        body,
        grid_spec=grid_spec,
        out_shape=jax.ShapeDtypeStruct((N, D), jnp.float32),
    )(n0a, xs, agg2, W1, b1, W2, b2, cWx, cWh, cb)


def kernel(xs, k_batch, bipartites_list,
           c1_W1, c1_b1, c1_W2, c1_b2, c1_cW, c1_cb,
           c2_W1, c2_b1, c2_W2, c2_b2, c2_cW, c2_cb):
    e0 = bipartites_list[0, 0].astype(jnp.int32)
    e1 = bipartites_list[0, 1].astype(jnp.int32)
    n0 = jnp.sum(k_batch == 0).astype(jnp.int32)
    n1 = jnp.int32(N) - n0
    zrows = jnp.zeros((STRIPE, D), jnp.float32)
    n0a = n0.reshape(1)
    gpad = jnp.zeros((EPAD - E,), jnp.int32)
    spad = jnp.full((EPAD - E,), SENT, jnp.int32)

    gidx1 = n0 + jnp.minimum(e1, n1 - 1)
    gidx1 = jnp.where(gidx1 < 0, gidx1 + N, gidx1)
    sidx1 = jnp.where(e0 < n0, e0, SENT)
    gidx1 = jnp.concatenate([gidx1, gpad]).reshape(NW, NB, K)
    sidx1 = jnp.concatenate([sidx1, spad]).reshape(NW, NB, K)
    acc1 = _sc_segment_sum(xs, gidx1, sidx1, zrows)
    xs1, cand0 = _combine_pass1(n0a, xs, acc1[0], acc1[1],
                                c1_W1, c1_b1.reshape(1, D),
                                c1_W2, c1_b2.reshape(1, D),
                                c1_cW[:D], c1_cW[D:], c1_cb.reshape(1, D))

    gidx2 = jnp.minimum(e0, n0 - 1)
    gidx2 = jnp.where(gidx2 < 0, gidx2 + N, gidx2)
    sidx2 = jnp.where(e1 < n1, e1, SENT)
    gidx2 = jnp.concatenate([gidx2, gpad]).reshape(NW, NB, K)
    sidx2 = jnp.concatenate([sidx2, spad]).reshape(NW, NB, K)
    acc2 = _sc_segment_sum(cand0, gidx2, sidx2, zrows)
    agg2 = lax.dynamic_update_slice(
        jnp.zeros((N + ACC, D), jnp.float32),
        acc2[0, :9500] + acc2[1, :9500], (n0, jnp.int32(0)))[:N]
    xs2 = _combine_pass2(n0a, xs1, agg2,
                         c2_W1, c2_b1.reshape(1, D),
                         c2_W2, c2_b2.reshape(1, D),
                         c2_cW[:D], c2_cW[D:], c2_cb.reshape(1, D))
    return xs2


# speedup vs baseline: 3.0592x; 3.0592x over previous
import functools

import jax
import jax.numpy as jnp
from jax import lax
from jax.experimental import pallas as pl
from jax.experimental.pallas import tpu as pltpu
from jax.experimental.pallas import tpu_sc as plsc

D = 128
N = 20000
E = 320000
NC = 2
NS = 16
NW = NC * NS
K = 128
NB = 80
EW = NB * K
EPAD = NW * EW
ACC = 9600
SENT = 9599
STRIPE = ACC // NS
BR = 400
NBLK = N // BR
AB = ACC // BR


def _sc_segment_sum(table, gidx, sidx, zrows):
    mesh = plsc.VectorSubcoreMesh(
        core_axis_name="c", subcore_axis_name="s",
        num_cores=NC, num_subcores=NS)

    @functools.partial(
        pl.kernel,
        out_type=jax.ShapeDtypeStruct((NC, ACC, D), jnp.float32),
        mesh=mesh,
        scratch_types=[
            pltpu.VMEM((NB, K), jnp.int32),
            pltpu.VMEM((NB, K), jnp.int32),
            pltpu.VMEM((K, D), jnp.float32),
            pltpu.VMEM((K, D), jnp.float32),
            pltpu.VMEM_SHARED((ACC, D), jnp.float32),
            pltpu.SemaphoreType.DMA,
            pltpu.SemaphoreType.DMA,
        ],
    )
    def seg_kernel(table_h, gidx_h, sidx_h, z_h, out_h,
                   gall, sall, rowsA, rowsB, acc, semA, semB):
        cid = lax.axis_index("c")
        sid = lax.axis_index("s")
        wid = sid * NC + cid
        pltpu.sync_copy(gidx_h.at[wid], gall)
        pltpu.sync_copy(sidx_h.at[wid], sall)
        pltpu.async_copy(table_h.at[gall.at[0]], rowsA, semA)
        pltpu.sync_copy(z_h, acc.at[pl.ds(sid * STRIPE, STRIPE)])
        plsc.subcore_barrier()

        def waitA(b):
            pltpu.make_async_copy(table_h.at[gall.at[b]], rowsA, semA).wait()

        def waitB(b):
            pltpu.make_async_copy(table_h.at[gall.at[b]], rowsB, semB).wait()

        def body(t, carry):
            b0 = 2 * t
            b1 = b0 + 1
            pltpu.async_copy(table_h.at[gall.at[b1]], rowsB, semB)
            waitA(b0)
            pltpu.sync_copy(rowsA, acc.at[sall.at[b0]], add=True)
            pltpu.async_copy(table_h.at[gall.at[b0 + 2]], rowsA, semA)
            waitB(b1)
            pltpu.sync_copy(rowsB, acc.at[sall.at[b1]], add=True)
            return carry

        lax.fori_loop(0, NB // 2 - 1, body, 0)
        pltpu.async_copy(table_h.at[gall.at[NB - 1]], rowsB, semB)
        waitA(NB - 2)
        pltpu.sync_copy(rowsA, acc.at[sall.at[NB - 2]], add=True)
        waitB(NB - 1)
        pltpu.sync_copy(rowsB, acc.at[sall.at[NB - 1]], add=True)
        plsc.subcore_barrier()
        pltpu.sync_copy(acc.at[pl.ds(sid * STRIPE, STRIPE)],
                        out_h.at[cid, pl.ds(sid * STRIPE, STRIPE)])

    return seg_kernel(table, gidx, sidx, zrows)


def _mlp_combine(x, W1, b1, W2, b2, cWx, cWh, cb, agg):
    h = jnp.maximum(jnp.dot(agg, W1, preferred_element_type=jnp.float32) + b1, 0.0)
    h = jnp.maximum(jnp.dot(h, W2, preferred_element_type=jnp.float32) + b2, 0.0)
    cand = jnp.dot(x, cWx, preferred_element_type=jnp.float32)
    cand = cand + jnp.dot(h, cWh, preferred_element_type=jnp.float32) + cb
    return jnp.maximum(cand, 0.0)


def _combine_pass1(n0a, xs, accA, accB, W1, b1, W2, b2, cWx, cWh, cb):
    def body(n0s, xsr, aAr, aBr, W1r, b1r, W2r, b2r, cWxr, cWhr, cbr,
             outr, candr):
        i = pl.program_id(0)
        rows = i * BR + lax.broadcasted_iota(jnp.int32, (BR, 1), 0)
        agg = jnp.where(rows < 9500, aAr[...] + aBr[...], 0.0)
        cand = _mlp_combine(xsr[...], W1r[...], b1r[...], W2r[...], b2r[...],
                            cWxr[...], cWhr[...], cbr[...], agg)
        candr[...] = cand
        outr[...] = jnp.where(rows < n0s[0], cand, xsr[...])

    w = lambda i, s: (0, 0)
    grid_spec = pltpu.PrefetchScalarGridSpec(
        num_scalar_prefetch=1,
        grid=(NBLK,),
        in_specs=[
            pl.BlockSpec((BR, D), lambda i, s: (i, 0)),
            pl.BlockSpec((BR, D), lambda i, s: (jnp.minimum(i, AB - 1), 0)),
            pl.BlockSpec((BR, D), lambda i, s: (jnp.minimum(i, AB - 1), 0)),
            pl.BlockSpec((D, D), w), pl.BlockSpec((1, D), w),
            pl.BlockSpec((D, D), w), pl.BlockSpec((1, D), w),
            pl.BlockSpec((D, D), w), pl.BlockSpec((D, D), w),
            pl.BlockSpec((1, D), w),
        ],
        out_specs=[pl.BlockSpec((BR, D), lambda i, s: (i, 0)),
                   pl.BlockSpec((BR, D), lambda i, s: (i, 0))],
    )
    return pl.pallas_call(
        body,
        grid_spec=grid_spec,
        out_shape=[jax.ShapeDtypeStruct((N, D), jnp.float32),
                   jax.ShapeDtypeStruct((N, D), jnp.float32)],
    )(n0a, xs, accA, accB, W1, b1, W2, b2, cWx, cWh, cb)


def _combine_pass2(n0a, xs, agg2, W1, b1, W2, b2, cWx, cWh, cb):
    def body(n0s, xsr, aggr, W1r, b1r, W2r, b2r, cWxr, cWhr, cbr, outr):
        i = pl.program_id(0)
        rows = i * BR + lax.broadcasted_iota(jnp.int32, (BR, 1), 0)
        cand = _mlp_combine(xsr[...], W1r[...], b1r[...], W2r[...], b2r[...],
                            cWxr[...], cWhr[...], cbr[...], aggr[...])
        outr[...] = jnp.where(rows >= n0s[0], cand, xsr[...])

    w = lambda i, s: (0, 0)
    grid_spec = pltpu.PrefetchScalarGridSpec(
        num_scalar_prefetch=1,
        grid=(NBLK,),
        in_specs=[
            pl.BlockSpec((BR, D), lambda i, s: (i, 0)),
            pl.BlockSpec((BR, D), lambda i, s: (i, 0)),
            pl.BlockSpec((D, D), w), pl.BlockSpec((1, D), w),
            pl.BlockSpec((D, D), w), pl.BlockSpec((1, D), w),
            pl.BlockSpec((D, D), w), pl.BlockSpec((D, D), w),
            pl.BlockSpec((1, D), w),
        ],
        out_specs=pl.BlockSpec((BR, D), lambda i, s: (i, 0)),
    )
    return pl.pallas_call(
        body,
        grid_spec=grid_spec,
        out_shape=jax.ShapeDtypeStruct((N, D), jnp.float32),
    )(n0a, xs, agg2, W1, b1, W2, b2, cWx, cWh, cb)


def kernel(xs, k_batch, bipartites_list,
           c1_W1, c1_b1, c1_W2, c1_b2, c1_cW, c1_cb,
           c2_W1, c2_b1, c2_W2, c2_b2, c2_cW, c2_cb):
    e0 = bipartites_list[0, 0].astype(jnp.int32)
    e1 = bipartites_list[0, 1].astype(jnp.int32)
    n0 = jnp.sum(k_batch == 0).astype(jnp.int32)
    n1 = jnp.int32(N) - n0
    zrows = jnp.zeros((STRIPE, D), jnp.float32)
    n0a = n0.reshape(1)
    gpad = jnp.arange(EPAD - E, dtype=jnp.int32) % jnp.int32(9500)
    spad = jnp.int32(9504) + jnp.arange(EPAD - E, dtype=jnp.int32) % jnp.int32(96)

    gidx1 = n0 + jnp.minimum(e1, n1 - 1)
    gidx1 = jnp.where(gidx1 < 0, gidx1 + N, gidx1)
    sidx1 = jnp.where(e0 < n0, e0, SENT)
    gidx1 = jnp.concatenate([gidx1, gpad]).reshape(NW, NB, K)
    sidx1 = jnp.concatenate([sidx1, spad]).reshape(NW, NB, K)
    acc1 = _sc_segment_sum(xs, gidx1, sidx1, zrows)
    xs1, cand0 = _combine_pass1(n0a, xs, acc1[0], acc1[1],
                                c1_W1, c1_b1.reshape(1, D),
                                c1_W2, c1_b2.reshape(1, D),
                                c1_cW[:D], c1_cW[D:], c1_cb.reshape(1, D))

    gidx2 = jnp.minimum(e0, n0 - 1)
    gidx2 = jnp.where(gidx2 < 0, gidx2 + N, gidx2)
    sidx2 = jnp.where(e1 < n1, e1, SENT)
    gidx2 = jnp.concatenate([gidx2, gpad]).reshape(NW, NB, K)
    sidx2 = jnp.concatenate([sidx2, spad]).reshape(NW, NB, K)
    acc2 = _sc_segment_sum(cand0, gidx2, sidx2, zrows)
    agg2 = lax.dynamic_update_slice(
        jnp.zeros((N + ACC, D), jnp.float32),
        acc2[0, :9500] + acc2[1, :9500], (n0, jnp.int32(0)))[:N]
    xs2 = _combine_pass2(n0a, xs1, agg2,
                         c2_W1, c2_b1.reshape(1, D),
                         c2_W2, c2_b2.reshape(1, D),
                         c2_cW[:D], c2_cW[D:], c2_cb.reshape(1, D))
    return xs2


# speedup vs baseline: 3.3341x; 1.0899x over previous
import functools

import jax
import jax.numpy as jnp
from jax import lax
from jax.experimental import pallas as pl
from jax.experimental.pallas import tpu as pltpu
from jax.experimental.pallas import tpu_sc as plsc

D = 128
N = 20000
E = 320000
NC = 2
NS = 16
NW = NC * NS
K = 128
NB = 80
EW = NB * K
EPAD = NW * EW
ACC = 9600
JBASE = 9504
STRIPE = ACC // NS
BR = 800
NBLK = N // BR
AB = ACC // BR


def _sc_segment_sum(table, gidx, sidx, zrows):
    mesh = plsc.VectorSubcoreMesh(
        core_axis_name="c", subcore_axis_name="s",
        num_cores=NC, num_subcores=NS)

    @functools.partial(
        pl.kernel,
        out_type=jax.ShapeDtypeStruct((NC, ACC, D), jnp.float32),
        mesh=mesh,
        scratch_types=[
            pltpu.VMEM((NB, K), jnp.int32),
            pltpu.VMEM((NB, K), jnp.int32),
            pltpu.VMEM((K, D), jnp.float32),
            pltpu.VMEM((K, D), jnp.float32),
            pltpu.VMEM_SHARED((ACC, D), jnp.float32),
            pltpu.SemaphoreType.DMA,
            pltpu.SemaphoreType.DMA,
        ],
    )
    def seg_kernel(table_h, gidx_h, sidx_h, z_h, out_h,
                   gall, sall, rowsA, rowsB, acc, semA, semB):
        cid = lax.axis_index("c")
        sid = lax.axis_index("s")
        wid = sid * NC + cid
        pltpu.sync_copy(gidx_h.at[wid], gall)
        pltpu.sync_copy(sidx_h.at[wid], sall)
        pltpu.async_copy(table_h.at[gall.at[0]], rowsA, semA)
        pltpu.sync_copy(z_h, acc.at[pl.ds(sid * STRIPE, STRIPE)])
        plsc.subcore_barrier()

        def waitA(b):
            pltpu.make_async_copy(table_h.at[gall.at[b]], rowsA, semA).wait()

        def waitB(b):
            pltpu.make_async_copy(table_h.at[gall.at[b]], rowsB, semB).wait()

        def body(t, carry):
            b0 = 2 * t
            b1 = b0 + 1
            pltpu.async_copy(table_h.at[gall.at[b1]], rowsB, semB)
            waitA(b0)
            pltpu.sync_copy(rowsA, acc.at[sall.at[b0]], add=True)
            pltpu.async_copy(table_h.at[gall.at[b0 + 2]], rowsA, semA)
            waitB(b1)
            pltpu.sync_copy(rowsB, acc.at[sall.at[b1]], add=True)
            return carry

        lax.fori_loop(0, NB // 2 - 1, body, 0)
        pltpu.async_copy(table_h.at[gall.at[NB - 1]], rowsB, semB)
        waitA(NB - 2)
        pltpu.sync_copy(rowsA, acc.at[sall.at[NB - 2]], add=True)
        waitB(NB - 1)
        pltpu.sync_copy(rowsB, acc.at[sall.at[NB - 1]], add=True)
        plsc.subcore_barrier()
        pltpu.sync_copy(acc.at[pl.ds(sid * STRIPE, STRIPE)],
                        out_h.at[cid, pl.ds(sid * STRIPE, STRIPE)])

    return seg_kernel(table, gidx, sidx, zrows)


def _mlp_combine(x, W1, b1, W2, b2, cWx, cWh, cb, agg):
    bf = jnp.bfloat16
    h = jnp.maximum(
        jnp.dot(agg.astype(bf), W1.astype(bf),
                preferred_element_type=jnp.float32) + b1, 0.0)
    h = jnp.maximum(
        jnp.dot(h.astype(bf), W2.astype(bf),
                preferred_element_type=jnp.float32) + b2, 0.0)
    cand = jnp.dot(x.astype(bf), cWx.astype(bf),
                   preferred_element_type=jnp.float32)
    cand = cand + jnp.dot(h.astype(bf), cWh.astype(bf),
                          preferred_element_type=jnp.float32) + cb
    return jnp.maximum(cand, 0.0)


def _combine_pass1(n0a, xs, accA, accB, W1, b1, W2, b2, cWx, cWh, cb):
    def body(n0s, xsr, aAr, aBr, W1r, b1r, W2r, b2r, cWxr, cWhr, cbr,
             outr, candr):
        i = pl.program_id(0)
        rows = i * BR + lax.broadcasted_iota(jnp.int32, (BR, 1), 0)
        agg = jnp.where(rows < 9500, aAr[...] + aBr[...], 0.0)
        cand = _mlp_combine(xsr[...], W1r[...], b1r[...], W2r[...], b2r[...],
                            cWxr[...], cWhr[...], cbr[...], agg)
        candr[...] = cand
        outr[...] = jnp.where(rows < n0s[0], cand, xsr[...])

    w = lambda i, s: (0, 0)
    grid_spec = pltpu.PrefetchScalarGridSpec(
        num_scalar_prefetch=1,
        grid=(NBLK,),
        in_specs=[
            pl.BlockSpec((BR, D), lambda i, s: (i, 0)),
            pl.BlockSpec((BR, D), lambda i, s: (jnp.minimum(i, AB - 1), 0)),
            pl.BlockSpec((BR, D), lambda i, s: (jnp.minimum(i, AB - 1), 0)),
            pl.BlockSpec((D, D), w), pl.BlockSpec((1, D), w),
            pl.BlockSpec((D, D), w), pl.BlockSpec((1, D), w),
            pl.BlockSpec((D, D), w), pl.BlockSpec((D, D), w),
            pl.BlockSpec((1, D), w),
        ],
        out_specs=[pl.BlockSpec((BR, D), lambda i, s: (i, 0)),
                   pl.BlockSpec((BR, D), lambda i, s: (i, 0))],
    )
    return pl.pallas_call(
        body,
        grid_spec=grid_spec,
        out_shape=[jax.ShapeDtypeStruct((N, D), jnp.float32),
                   jax.ShapeDtypeStruct((N, D), jnp.float32)],
    )(n0a, xs, accA, accB, W1, b1, W2, b2, cWx, cWh, cb)


def _combine_pass2(n0a, xs, agg2, W1, b1, W2, b2, cWx, cWh, cb):
    def body(n0s, xsr, aggr, W1r, b1r, W2r, b2r, cWxr, cWhr, cbr, outr):
        i = pl.program_id(0)
        rows = i * BR + lax.broadcasted_iota(jnp.int32, (BR, 1), 0)
        cand = _mlp_combine(xsr[...], W1r[...], b1r[...], W2r[...], b2r[...],
                            cWxr[...], cWhr[...], cbr[...], aggr[...])
        outr[...] = jnp.where(rows >= n0s[0], cand, xsr[...])

    w = lambda i, s: (0, 0)
    grid_spec = pltpu.PrefetchScalarGridSpec(
        num_scalar_prefetch=1,
        grid=(NBLK,),
        in_specs=[
            pl.BlockSpec((BR, D), lambda i, s: (i, 0)),
            pl.BlockSpec((BR, D), lambda i, s: (i, 0)),
            pl.BlockSpec((D, D), w), pl.BlockSpec((1, D), w),
            pl.BlockSpec((D, D), w), pl.BlockSpec((1, D), w),
            pl.BlockSpec((D, D), w), pl.BlockSpec((D, D), w),
            pl.BlockSpec((1, D), w),
        ],
        out_specs=pl.BlockSpec((BR, D), lambda i, s: (i, 0)),
    )
    return pl.pallas_call(
        body,
        grid_spec=grid_spec,
        out_shape=jax.ShapeDtypeStruct((N, D), jnp.float32),
    )(n0a, xs, agg2, W1, b1, W2, b2, cWx, cWh, cb)


def kernel(xs, k_batch, bipartites_list,
           c1_W1, c1_b1, c1_W2, c1_b2, c1_cW, c1_cb,
           c2_W1, c2_b1, c2_W2, c2_b2, c2_cW, c2_cb):
    e0 = bipartites_list[0, 0].astype(jnp.int32)
    e1 = bipartites_list[0, 1].astype(jnp.int32)
    n0 = jnp.sum(k_batch == 0).astype(jnp.int32)
    n1 = jnp.int32(N) - n0
    zrows = jnp.zeros((STRIPE, D), jnp.float32)
    n0a = n0.reshape(1)
    ar = jnp.arange(EPAD - E, dtype=jnp.int32)
    gpad = ar % jnp.int32(9500)
    spad = jnp.int32(JBASE) + ar % jnp.int32(96)

    gidx1 = n0 + jnp.minimum(e1, n1 - 1)
    gidx1 = jnp.where(gidx1 < 0, gidx1 + N, gidx1)
    sidx1 = jnp.where(e0 < n0, e0, JBASE + e0 % jnp.int32(96))
    gidx1 = jnp.concatenate([gidx1, gpad]).reshape(NW, NB, K)
    sidx1 = jnp.concatenate([sidx1, spad]).reshape(NW, NB, K)
    acc1 = _sc_segment_sum(xs, gidx1, sidx1, zrows)
    xs1, cand0 = _combine_pass1(n0a, xs, acc1[0], acc1[1],
                                c1_W1, c1_b1.reshape(1, D),
                                c1_W2, c1_b2.reshape(1, D),
                                c1_cW[:D], c1_cW[D:], c1_cb.reshape(1, D))

    gidx2 = jnp.minimum(e0, n0 - 1)
    gidx2 = jnp.where(gidx2 < 0, gidx2 + N, gidx2)
    sidx2 = jnp.where(e1 < n1, e1, JBASE + e1 % jnp.int32(96))
    gidx2 = jnp.concatenate([gidx2, gpad]).reshape(NW, NB, K)
    sidx2 = jnp.concatenate([sidx2, spad]).reshape(NW, NB, K)
    acc2 = _sc_segment_sum(cand0, gidx2, sidx2, zrows)
    agg2 = lax.dynamic_update_slice(
        jnp.zeros((N + ACC, D), jnp.float32),
        acc2[0, :9500] + acc2[1, :9500], (n0, jnp.int32(0)))[:N]
    xs2 = _combine_pass2(n0a, xs1, agg2,
                         c2_W1, c2_b1.reshape(1, D),
                         c2_W2, c2_b2.reshape(1, D),
                         c2_cW[:D], c2_cW[D:], c2_cb.reshape(1, D))
    return xs2


# speedup vs baseline: 3.8794x; 1.1636x over previous
import functools

import jax
import jax.numpy as jnp
from jax import lax
from jax.experimental import pallas as pl
from jax.experimental.pallas import tpu as pltpu
from jax.experimental.pallas import tpu_sc as plsc

D = 128
N = 20000
E = 320000
NC = 2
NS = 16
NW = NC * NS
K = 128
NB = 80
EW = NB * K
EPAD = NW * EW
ACC = 9728
JBASE = 9600
STRIPE = ACC // NS
BR = 800
NBLK = N // BR
AB = 12
CH = 80


def _prep_indices(k_batch_pad, edges):
    ER = E // D
    PR = (EPAD - E) // D

    def body(kbr, edr, n0o, g1o, s1o, g2o, s2o):
        n0 = jnp.int32(N) - jnp.sum(kbr[...])
        n1 = jnp.int32(N) - n0
        e0 = edr[0]
        e1 = edr[1]
        junk0 = JBASE + (e0 & 127)
        junk1 = JBASE + (e1 & 127)
        g1 = n0 + jnp.minimum(e1, n1 - 1)
        g1 = jnp.where(g1 < 0, g1 + N, g1)
        s1 = jnp.where(e0 < n0, e0, junk0)
        g2 = jnp.minimum(e0, n0 - 1)
        g2 = jnp.where(g2 < 0, g2 + N, g2)
        r = n0 - (n0 // CH) * CH
        s2 = jnp.where(e1 < n1, e1 + r, junk1)
        flat = (lax.broadcasted_iota(jnp.int32, (PR, D), 0) * D
                + lax.broadcasted_iota(jnp.int32, (PR, D), 1))
        g1o[:ER] = g1
        s1o[:ER] = s1
        g2o[:ER] = g2
        s2o[:ER] = s2
        g1o[ER:] = flat
        s1o[ER:] = JBASE + (flat & 127)
        g2o[ER:] = flat
        s2o[ER:] = JBASE + (flat & 127)
        n0o[0, 0] = n0

    return pl.pallas_call(
        body,
        in_specs=[pl.BlockSpec((160, D), lambda: (0, 0)),
                  pl.BlockSpec((2, ER, D), lambda: (0, 0, 0))],
        out_specs=[pl.BlockSpec(memory_space=pltpu.SMEM),
                   pl.BlockSpec((ER + PR, D), lambda: (0, 0)),
                   pl.BlockSpec((ER + PR, D), lambda: (0, 0)),
                   pl.BlockSpec((ER + PR, D), lambda: (0, 0)),
                   pl.BlockSpec((ER + PR, D), lambda: (0, 0))],
        out_shape=[jax.ShapeDtypeStruct((1, 1), jnp.int32)]
        + [jax.ShapeDtypeStruct((ER + PR, D), jnp.int32)] * 4,
    )(k_batch_pad, edges)


def _sc_segment_sum(table, gidx, sidx, zrows):
    mesh = plsc.VectorSubcoreMesh(
        core_axis_name="c", subcore_axis_name="s",
        num_cores=NC, num_subcores=NS)

    @functools.partial(
        pl.kernel,
        out_type=jax.ShapeDtypeStruct((NC, ACC, D), jnp.float32),
        mesh=mesh,
        scratch_types=[
            pltpu.VMEM((NB, K), jnp.int32),
            pltpu.VMEM((NB, K), jnp.int32),
            pltpu.VMEM((K, D), jnp.float32),
            pltpu.VMEM((K, D), jnp.float32),
            pltpu.VMEM_SHARED((ACC, D), jnp.float32),
            pltpu.SemaphoreType.DMA,
            pltpu.SemaphoreType.DMA,
        ],
    )
    def seg_kernel(table_h, gidx_h, sidx_h, z_h, out_h,
                   gall, sall, rowsA, rowsB, acc, semA, semB):
        cid = lax.axis_index("c")
        sid = lax.axis_index("s")
        wid = sid * NC + cid
        pltpu.sync_copy(gidx_h.at[wid], gall)
        pltpu.sync_copy(sidx_h.at[wid], sall)
        pltpu.async_copy(table_h.at[gall.at[0]], rowsA, semA)
        pltpu.sync_copy(z_h, acc.at[pl.ds(sid * STRIPE, STRIPE)])
        plsc.subcore_barrier()

        def waitA(b):
            pltpu.make_async_copy(table_h.at[gall.at[b]], rowsA, semA).wait()

        def waitB(b):
            pltpu.make_async_copy(table_h.at[gall.at[b]], rowsB, semB).wait()

        def body(t, carry):
            b0 = 2 * t
            b1 = b0 + 1
            pltpu.async_copy(table_h.at[gall.at[b1]], rowsB, semB)
            waitA(b0)
            pltpu.sync_copy(rowsA, acc.at[sall.at[b0]], add=True)
            pltpu.async_copy(table_h.at[gall.at[b0 + 2]], rowsA, semA)
            waitB(b1)
            pltpu.sync_copy(rowsB, acc.at[sall.at[b1]], add=True)
            return carry

        lax.fori_loop(0, NB // 2 - 1, body, 0)
        pltpu.async_copy(table_h.at[gall.at[NB - 1]], rowsB, semB)
        waitA(NB - 2)
        pltpu.sync_copy(rowsA, acc.at[sall.at[NB - 2]], add=True)
        waitB(NB - 1)
        pltpu.sync_copy(rowsB, acc.at[sall.at[NB - 1]], add=True)
        plsc.subcore_barrier()
        pltpu.sync_copy(acc.at[pl.ds(sid * STRIPE, STRIPE)],
                        out_h.at[cid, pl.ds(sid * STRIPE, STRIPE)])

    return seg_kernel(table, gidx, sidx, zrows)


def _mlp_combine(x, W1, b1, W2, b2, cWx, cWh, cb, agg):
    bf = jnp.bfloat16
    h = jnp.maximum(
        jnp.dot(agg.astype(bf), W1.astype(bf),
                preferred_element_type=jnp.float32) + b1, 0.0)
    h = jnp.maximum(
        jnp.dot(h.astype(bf), W2.astype(bf),
                preferred_element_type=jnp.float32) + b2, 0.0)
    cand = jnp.dot(x.astype(bf), cWx.astype(bf),
                   preferred_element_type=jnp.float32)
    cand = cand + jnp.dot(h.astype(bf), cWh.astype(bf),
                          preferred_element_type=jnp.float32) + cb
    return jnp.maximum(cand, 0.0)


def _combine_pass1(n0a, xs, acc, W1, b1, W2, b2, cW, cb):
    def body(n0s, xsr, ar, W1r, b1r, W2r, b2r, cWr, cbr, outr, candr):
        i = pl.program_id(0)
        rows = i * BR + lax.broadcasted_iota(jnp.int32, (BR, 1), 0)
        agg = jnp.where(rows < 9500, ar[0] + ar[1], 0.0)
        cand = _mlp_combine(xsr[...], W1r[...], b1r[...], W2r[...], b2r[...],
                            cWr[:D], cWr[D:], cbr[...], agg)
        candr[...] = cand
        outr[...] = jnp.where(rows < n0s[0], cand, xsr[...])

    w = lambda i, s: (0, 0)
    grid_spec = pltpu.PrefetchScalarGridSpec(
        num_scalar_prefetch=1,
        grid=(NBLK,),
        in_specs=[
            pl.BlockSpec((BR, D), lambda i, s: (i, 0)),
            pl.BlockSpec((NC, BR, D),
                         lambda i, s: (0, jnp.minimum(i, AB - 1), 0)),
            pl.BlockSpec((D, D), w), pl.BlockSpec((1, D), w),
            pl.BlockSpec((D, D), w), pl.BlockSpec((1, D), w),
            pl.BlockSpec((2 * D, D), w), pl.BlockSpec((1, D), w),
        ],
        out_specs=[pl.BlockSpec((BR, D), lambda i, s: (i, 0)),
                   pl.BlockSpec((BR, D), lambda i, s: (i, 0))],
    )
    return pl.pallas_call(
        body,
        grid_spec=grid_spec,
        out_shape=[jax.ShapeDtypeStruct((N, D), jnp.float32),
                   jax.ShapeDtypeStruct((N, D), jnp.float32)],
    )(n0a, xs, acc, W1, b1, W2, b2, cW, cb)


def _combine_pass2(n0a, xs, acc, W1, b1, W2, b2, cW, cb):
    def body(n0s, xsr, ar, W1r, b1r, W2r, b2r, cWr, cbr, outr):
        i = pl.program_id(0)
        rows = i * BR + lax.broadcasted_iota(jnp.int32, (BR, 1), 0)
        n0v = n0s[0]
        w0 = i * BR - (n0v // CH) * CH
        chunks = []
        for c in range(BR // CH):
            s = jnp.clip(w0 + c * CH, 0, ACC - CH)
            s = pl.multiple_of(s, 8)
            chunks.append(ar[0, pl.ds(s, CH), :] + ar[1, pl.ds(s, CH), :])
        agg = jnp.concatenate(chunks, axis=0)
        agg = jnp.where((rows >= n0v) & (rows < n0v + 9500), agg, 0.0)
        cand = _mlp_combine(xsr[...], W1r[...], b1r[...], W2r[...], b2r[...],
                            cWr[:D], cWr[D:], cbr[...], agg)
        outr[...] = jnp.where(rows >= n0v, cand, xsr[...])

    w = lambda i, s: (0, 0)
    grid_spec = pltpu.PrefetchScalarGridSpec(
        num_scalar_prefetch=1,
        grid=(NBLK,),
        in_specs=[
            pl.BlockSpec((BR, D), lambda i, s: (i, 0)),
            pl.BlockSpec((NC, ACC, D), lambda i, s: (0, 0, 0)),
            pl.BlockSpec((D, D), w), pl.BlockSpec((1, D), w),
            pl.BlockSpec((D, D), w), pl.BlockSpec((1, D), w),
            pl.BlockSpec((2 * D, D), w), pl.BlockSpec((1, D), w),
        ],
        out_specs=pl.BlockSpec((BR, D), lambda i, s: (i, 0)),
    )
    return pl.pallas_call(
        body,
        grid_spec=grid_spec,
        out_shape=jax.ShapeDtypeStruct((N, D), jnp.float32),
    )(n0a, xs, acc, W1, b1, W2, b2, cW, cb)


def kernel(xs, k_batch, bipartites_list,
           c1_W1, c1_b1, c1_W2, c1_b2, c1_cW, c1_cb,
           c2_W1, c2_b1, c2_W2, c2_b2, c2_cW, c2_cb):
    kbp = jnp.concatenate(
        [k_batch.astype(jnp.int32), jnp.zeros((480,), jnp.int32)]
    ).reshape(160, D)
    edges = bipartites_list.astype(jnp.int32).reshape(2, E // D, D)
    n0o, g1, s1, g2, s2 = _prep_indices(kbp, edges)
    n0a = n0o[0]
    zrows = jnp.zeros((STRIPE, D), jnp.float32)

    acc1 = _sc_segment_sum(xs, g1.reshape(NW, NB, K), s1.reshape(NW, NB, K),
                           zrows)
    xs1, cand0 = _combine_pass1(n0a, xs, acc1,
                                c1_W1, c1_b1.reshape(1, D),
                                c1_W2, c1_b2.reshape(1, D),
                                c1_cW, c1_cb.reshape(1, D))

    acc2 = _sc_segment_sum(cand0, g2.reshape(NW, NB, K),
                           s2.reshape(NW, NB, K), zrows)
    xs2 = _combine_pass2(n0a, xs1, acc2,
                         c2_W1, c2_b1.reshape(1, D),
                         c2_W2, c2_b2.reshape(1, D),
                         c2_cW, c2_cb.reshape(1, D))
    return xs2


# speedup vs baseline: 3.9244x; 1.0116x over previous
import functools

import jax
import jax.numpy as jnp
from jax import lax
from jax.experimental import pallas as pl
from jax.experimental.pallas import tpu as pltpu
from jax.experimental.pallas import tpu_sc as plsc

D = 128
N = 20000
E = 320000
NC = 2
NS = 16
NW = NC * NS
K = 128
NB = 80
EW = NB * K
EPAD = NW * EW
ACC = 9728
JBASE = 9600
STRIPE = ACC // NS
BR = 800
NBLK = N // BR
AB = 12
CH = 80


def _prep_indices(k_batch_pad, edges):
    ER = E // D
    PR = (EPAD - E) // D

    def body(kbr, edr, n0o, g1o, s1o, g2o, s2o):
        n0 = jnp.int32(N) - jnp.sum(kbr[...])
        n1 = jnp.int32(N) - n0
        e0 = edr[0]
        e1 = edr[1]
        junk0 = JBASE + (e0 & 127)
        junk1 = JBASE + (e1 & 127)
        g1 = n0 + jnp.minimum(e1, n1 - 1)
        g1 = jnp.where(g1 < 0, g1 + N, g1)
        s1 = jnp.where(e0 < n0, e0, junk0)
        g2 = jnp.minimum(e0, n0 - 1)
        g2 = jnp.where(g2 < 0, g2 + N, g2)
        r = n0 - (n0 // CH) * CH
        s2 = jnp.where(e1 < n1, e1 + r, junk1)
        flat = (lax.broadcasted_iota(jnp.int32, (PR, D), 0) * D
                + lax.broadcasted_iota(jnp.int32, (PR, D), 1))
        g1o[:ER] = g1
        s1o[:ER] = s1
        g2o[:ER] = g2
        s2o[:ER] = s2
        g1o[ER:] = flat
        s1o[ER:] = JBASE + (flat & 127)
        g2o[ER:] = flat
        s2o[ER:] = JBASE + (flat & 127)
        n0o[0, 0] = n0

    return pl.pallas_call(
        body,
        in_specs=[pl.BlockSpec((160, D), lambda: (0, 0)),
                  pl.BlockSpec((2, ER, D), lambda: (0, 0, 0))],
        out_specs=[pl.BlockSpec(memory_space=pltpu.SMEM),
                   pl.BlockSpec((ER + PR, D), lambda: (0, 0)),
                   pl.BlockSpec((ER + PR, D), lambda: (0, 0)),
                   pl.BlockSpec((ER + PR, D), lambda: (0, 0)),
                   pl.BlockSpec((ER + PR, D), lambda: (0, 0))],
        out_shape=[jax.ShapeDtypeStruct((1, 1), jnp.int32)]
        + [jax.ShapeDtypeStruct((ER + PR, D), jnp.int32)] * 4,
    )(k_batch_pad, edges)


def _sc_segment_sum(table, gidx, sidx, zrows):
    mesh = plsc.VectorSubcoreMesh(
        core_axis_name="c", subcore_axis_name="s",
        num_cores=NC, num_subcores=NS)

    @functools.partial(
        pl.kernel,
        out_type=jax.ShapeDtypeStruct((NC, ACC, D), jnp.float32),
        mesh=mesh,
        scratch_types=[
            pltpu.VMEM((NB, K), jnp.int32),
            pltpu.VMEM((NB, K), jnp.int32),
            pltpu.VMEM((K, D), jnp.float32),
            pltpu.VMEM((K, D), jnp.float32),
            pltpu.VMEM_SHARED((ACC, D), jnp.float32),
            pltpu.SemaphoreType.DMA,
            pltpu.SemaphoreType.DMA,
        ],
    )
    def seg_kernel(table_h, gidx_h, sidx_h, z_h, out_h,
                   gall, sall, rowsA, rowsB, acc, semA, semB):
        cid = lax.axis_index("c")
        sid = lax.axis_index("s")
        wid = sid * NC + cid
        pltpu.sync_copy(gidx_h.at[wid], gall)
        pltpu.sync_copy(sidx_h.at[wid], sall)
        pltpu.async_copy(table_h.at[gall.at[0]], rowsA, semA)
        pltpu.sync_copy(z_h, acc.at[pl.ds(sid * STRIPE, STRIPE)])
        plsc.subcore_barrier()

        def waitA(b):
            pltpu.make_async_copy(table_h.at[gall.at[b]], rowsA, semA).wait()

        def waitB(b):
            pltpu.make_async_copy(table_h.at[gall.at[b]], rowsB, semB).wait()

        def body(t, carry):
            b0 = 2 * t
            b1 = b0 + 1
            pltpu.async_copy(table_h.at[gall.at[b1]], rowsB, semB)
            waitA(b0)
            pltpu.sync_copy(rowsA, acc.at[sall.at[b0]], add=True)
            pltpu.async_copy(table_h.at[gall.at[b0 + 2]], rowsA, semA)
            waitB(b1)
            pltpu.sync_copy(rowsB, acc.at[sall.at[b1]], add=True)
            return carry

        lax.fori_loop(0, NB // 2 - 1, body, 0)
        pltpu.async_copy(table_h.at[gall.at[NB - 1]], rowsB, semB)
        waitA(NB - 2)
        pltpu.sync_copy(rowsA, acc.at[sall.at[NB - 2]], add=True)
        waitB(NB - 1)
        pltpu.sync_copy(rowsB, acc.at[sall.at[NB - 1]], add=True)
        plsc.subcore_barrier()
        pltpu.sync_copy(acc.at[pl.ds(sid * STRIPE, STRIPE)],
                        out_h.at[cid, pl.ds(sid * STRIPE, STRIPE)])

    return seg_kernel(table, gidx, sidx, zrows)


def _mlp_combine(x, W1, b1, W2, b2, cWx, cWh, cb, agg):
    bf = jnp.bfloat16
    h = jnp.maximum(
        jnp.dot(agg.astype(bf), W1.astype(bf),
                preferred_element_type=jnp.float32) + b1, 0.0)
    h = jnp.maximum(
        jnp.dot(h.astype(bf), W2.astype(bf),
                preferred_element_type=jnp.float32) + b2, 0.0)
    cand = jnp.dot(x.astype(bf), cWx.astype(bf),
                   preferred_element_type=jnp.float32)
    cand = cand + jnp.dot(h.astype(bf), cWh.astype(bf),
                          preferred_element_type=jnp.float32) + cb
    return jnp.maximum(cand, 0.0)


def _combine_pass1(n0a, xs, acc, W1, b1, W2, b2, cW, cb):
    def body(n0s, xsr, ar, W1r, b1r, W2r, b2r, cWr, cbr, outr, candr):
        i = pl.program_id(0)
        rows = i * BR + lax.broadcasted_iota(jnp.int32, (BR, 1), 0)
        n0v = n0s[0]
        do = (i * BR < n0v) | (n0v == 0)

        @pl.when(do)
        def _():
            agg = jnp.where(rows < 9500, ar[0] + ar[1], 0.0)
            cand = _mlp_combine(xsr[...], W1r[...], b1r[...], W2r[...],
                                b2r[...], cWr[:D], cWr[D:], cbr[...], agg)
            candr[...] = cand
            outr[...] = jnp.where(rows < n0v, cand, xsr[...])

        @pl.when(jnp.logical_not(do))
        def _():
            candr[...] = xsr[...]
            outr[...] = xsr[...]

    w = lambda i, s: (0, 0)
    grid_spec = pltpu.PrefetchScalarGridSpec(
        num_scalar_prefetch=1,
        grid=(NBLK,),
        in_specs=[
            pl.BlockSpec((BR, D), lambda i, s: (i, 0)),
            pl.BlockSpec((NC, BR, D),
                         lambda i, s: (0, jnp.minimum(i, AB - 1), 0)),
            pl.BlockSpec((D, D), w), pl.BlockSpec((1, D), w),
            pl.BlockSpec((D, D), w), pl.BlockSpec((1, D), w),
            pl.BlockSpec((2 * D, D), w), pl.BlockSpec((1, D), w),
        ],
        out_specs=[pl.BlockSpec((BR, D), lambda i, s: (i, 0)),
                   pl.BlockSpec((BR, D), lambda i, s: (i, 0))],
    )
    return pl.pallas_call(
        body,
        grid_spec=grid_spec,
        out_shape=[jax.ShapeDtypeStruct((N, D), jnp.float32),
                   jax.ShapeDtypeStruct((N, D), jnp.float32)],
    )(n0a, xs, acc, W1, b1, W2, b2, cW, cb)


def _combine_pass2(n0a, xs, acc, W1, b1, W2, b2, cW, cb):
    def body(n0s, xsr, ar, W1r, b1r, W2r, b2r, cWr, cbr, outr):
        i = pl.program_id(0)
        rows = i * BR + lax.broadcasted_iota(jnp.int32, (BR, 1), 0)
        n0v = n0s[0]
        do = (i + 1) * BR > n0v

        @pl.when(do)
        def _():
            w0 = i * BR - (n0v // CH) * CH
            chunks = []
            for c in range(BR // CH):
                s = jnp.clip(w0 + c * CH, 0, ACC - CH)
                s = pl.multiple_of(s, 8)
                chunks.append(ar[0, pl.ds(s, CH), :] + ar[1, pl.ds(s, CH), :])
            agg = jnp.concatenate(chunks, axis=0)
            agg = jnp.where((rows >= n0v) & (rows < n0v + 9500), agg, 0.0)
            cand = _mlp_combine(xsr[...], W1r[...], b1r[...], W2r[...],
                                b2r[...], cWr[:D], cWr[D:], cbr[...], agg)
            outr[...] = jnp.where(rows >= n0v, cand, xsr[...])

        @pl.when(jnp.logical_not(do))
        def _():
            outr[...] = xsr[...]

    w = lambda i, s: (0, 0)
    grid_spec = pltpu.PrefetchScalarGridSpec(
        num_scalar_prefetch=1,
        grid=(NBLK,),
        in_specs=[
            pl.BlockSpec((BR, D), lambda i, s: (i, 0)),
            pl.BlockSpec((NC, ACC, D), lambda i, s: (0, 0, 0)),
            pl.BlockSpec((D, D), w), pl.BlockSpec((1, D), w),
            pl.BlockSpec((D, D), w), pl.BlockSpec((1, D), w),
            pl.BlockSpec((2 * D, D), w), pl.BlockSpec((1, D), w),
        ],
        out_specs=pl.BlockSpec((BR, D), lambda i, s: (i, 0)),
    )
    return pl.pallas_call(
        body,
        grid_spec=grid_spec,
        out_shape=jax.ShapeDtypeStruct((N, D), jnp.float32),
    )(n0a, xs, acc, W1, b1, W2, b2, cW, cb)


def kernel(xs, k_batch, bipartites_list,
           c1_W1, c1_b1, c1_W2, c1_b2, c1_cW, c1_cb,
           c2_W1, c2_b1, c2_W2, c2_b2, c2_cW, c2_cb):
    kbp = jnp.concatenate(
        [k_batch.astype(jnp.int32), jnp.zeros((480,), jnp.int32)]
    ).reshape(160, D)
    edges = bipartites_list.astype(jnp.int32).reshape(2, E // D, D)
    n0o, g1, s1, g2, s2 = _prep_indices(kbp, edges)
    n0a = n0o[0]
    zrows = jnp.zeros((STRIPE, D), jnp.float32)

    acc1 = _sc_segment_sum(xs, g1.reshape(NW, NB, K), s1.reshape(NW, NB, K),
                           zrows)
    xs1, cand0 = _combine_pass1(n0a, xs, acc1,
                                c1_W1, c1_b1.reshape(1, D),
                                c1_W2, c1_b2.reshape(1, D),
                                c1_cW, c1_cb.reshape(1, D))

    acc2 = _sc_segment_sum(cand0, g2.reshape(NW, NB, K),
                           s2.reshape(NW, NB, K), zrows)
    xs2 = _combine_pass2(n0a, xs1, acc2,
                         c2_W1, c2_b1.reshape(1, D),
                         c2_W2, c2_b2.reshape(1, D),
                         c2_cW, c2_cb.reshape(1, D))
    return xs2


# speedup vs baseline: 3.9248x; 1.0001x over previous
import functools

import jax
import jax.numpy as jnp
from jax import lax
from jax.experimental import pallas as pl
from jax.experimental.pallas import tpu as pltpu
from jax.experimental.pallas import tpu_sc as plsc

D = 128
N = 20000
E = 320000
NC = 2
NS = 16
NW = NC * NS
K = 128
NB = 80
EW = NB * K
EPAD = NW * EW
ACC = 9728
JBASE = 9600
STRIPE = ACC // NS
BR = 800
NBLK = N // BR
AB = 12
CH = 80


def _prep_indices(k_batch_pad, edges):
    ER = E // D
    PR = (EPAD - E) // D

    def body(kbr, edr, n0o, g1o, s1o, g2o, s2o):
        n0 = jnp.int32(N) - jnp.sum(kbr[...])
        n1 = jnp.int32(N) - n0
        e0 = edr[:ER]
        e1 = edr[ER:]
        junk0 = JBASE + (e0 & 127)
        junk1 = JBASE + (e1 & 127)
        g1 = n0 + jnp.minimum(e1, n1 - 1)
        g1 = jnp.where(g1 < 0, g1 + N, g1)
        s1 = jnp.where(e0 < n0, e0, junk0)
        g2 = jnp.minimum(e0, n0 - 1)
        g2 = jnp.where(g2 < 0, g2 + N, g2)
        r = n0 - (n0 // CH) * CH
        s2 = jnp.where(e1 < n1, e1 + r, junk1)
        flat = (lax.broadcasted_iota(jnp.int32, (PR, D), 0) * D
                + lax.broadcasted_iota(jnp.int32, (PR, D), 1))
        g1o[:ER] = g1
        s1o[:ER] = s1
        g2o[:ER] = g2
        s2o[:ER] = s2
        g1o[ER:] = flat
        s1o[ER:] = JBASE + (flat & 127)
        g2o[ER:] = flat
        s2o[ER:] = JBASE + (flat & 127)
        n0o[0, 0] = n0

    return pl.pallas_call(
        body,
        in_specs=[pl.BlockSpec((160, D), lambda: (0, 0)),
                  pl.BlockSpec((2 * ER, D), lambda: (0, 0))],
        out_specs=[pl.BlockSpec(memory_space=pltpu.SMEM),
                   pl.BlockSpec((ER + PR, D), lambda: (0, 0)),
                   pl.BlockSpec((ER + PR, D), lambda: (0, 0)),
                   pl.BlockSpec((ER + PR, D), lambda: (0, 0)),
                   pl.BlockSpec((ER + PR, D), lambda: (0, 0))],
        out_shape=[jax.ShapeDtypeStruct((1, 1), jnp.int32)]
        + [jax.ShapeDtypeStruct((ER + PR, D), jnp.int32)] * 4,
    )(k_batch_pad, edges)


def _sc_segment_sum(table, gidx, sidx, zrows):
    mesh = plsc.VectorSubcoreMesh(
        core_axis_name="c", subcore_axis_name="s",
        num_cores=NC, num_subcores=NS)

    @functools.partial(
        pl.kernel,
        out_type=jax.ShapeDtypeStruct((NC, ACC, D), jnp.float32),
        mesh=mesh,
        scratch_types=[
            pltpu.VMEM((NB, K), jnp.int32),
            pltpu.VMEM((NB, K), jnp.int32),
            pltpu.VMEM((K, D), jnp.float32),
            pltpu.VMEM((K, D), jnp.float32),
            pltpu.VMEM_SHARED((ACC, D), jnp.float32),
            pltpu.SemaphoreType.DMA,
            pltpu.SemaphoreType.DMA,
        ],
    )
    def seg_kernel(table_h, gidx_h, sidx_h, z_h, out_h,
                   gall, sall, rowsA, rowsB, acc, semA, semB):
        cid = lax.axis_index("c")
        sid = lax.axis_index("s")
        wid = sid * NC + cid
        pltpu.sync_copy(gidx_h.at[wid], gall)
        pltpu.sync_copy(sidx_h.at[wid], sall)
        pltpu.async_copy(table_h.at[gall.at[0]], rowsA, semA)
        pltpu.sync_copy(z_h, acc.at[pl.ds(sid * STRIPE, STRIPE)])
        plsc.subcore_barrier()

        def waitA(b):
            pltpu.make_async_copy(table_h.at[gall.at[b]], rowsA, semA).wait()

        def waitB(b):
            pltpu.make_async_copy(table_h.at[gall.at[b]], rowsB, semB).wait()

        def body(t, carry):
            b0 = 2 * t
            b1 = b0 + 1
            pltpu.async_copy(table_h.at[gall.at[b1]], rowsB, semB)
            waitA(b0)
            pltpu.sync_copy(rowsA, acc.at[sall.at[b0]], add=True)
            pltpu.async_copy(table_h.at[gall.at[b0 + 2]], rowsA, semA)
            waitB(b1)
            pltpu.sync_copy(rowsB, acc.at[sall.at[b1]], add=True)
            return carry

        lax.fori_loop(0, NB // 2 - 1, body, 0)
        pltpu.async_copy(table_h.at[gall.at[NB - 1]], rowsB, semB)
        waitA(NB - 2)
        pltpu.sync_copy(rowsA, acc.at[sall.at[NB - 2]], add=True)
        waitB(NB - 1)
        pltpu.sync_copy(rowsB, acc.at[sall.at[NB - 1]], add=True)
        plsc.subcore_barrier()
        pltpu.sync_copy(acc.at[pl.ds(sid * STRIPE, STRIPE)],
                        out_h.at[cid, pl.ds(sid * STRIPE, STRIPE)])

    return seg_kernel(table, gidx, sidx, zrows)


def _mlp_combine(x, W1, b1, W2, b2, cWx, cWh, cb, agg):
    bf = jnp.bfloat16
    h = jnp.maximum(
        jnp.dot(agg.astype(bf), W1.astype(bf),
                preferred_element_type=jnp.float32) + b1, 0.0)
    h = jnp.maximum(
        jnp.dot(h.astype(bf), W2.astype(bf),
                preferred_element_type=jnp.float32) + b2, 0.0)
    cand = jnp.dot(x.astype(bf), cWx.astype(bf),
                   preferred_element_type=jnp.float32)
    cand = cand + jnp.dot(h.astype(bf), cWh.astype(bf),
                          preferred_element_type=jnp.float32) + cb
    return jnp.maximum(cand, 0.0)


def _combine_pass1(n0a, xs, acc, W1, b1, W2, b2, cW, cb):
    def body(n0s, xsr, ar, W1r, b1r, W2r, b2r, cWr, cbr, outr, candr):
        i = pl.program_id(0)
        rows = i * BR + lax.broadcasted_iota(jnp.int32, (BR, 1), 0)
        n0v = n0s[0]
        do = (i * BR < n0v) | (n0v == 0)

        @pl.when(do)
        def _():
            agg = jnp.where(rows < 9500, ar[0] + ar[1], 0.0)
            cand = _mlp_combine(xsr[...], W1r[...], b1r[...], W2r[...],
                                b2r[...], cWr[:D], cWr[D:], cbr[...], agg)
            candr[...] = cand
            outr[...] = jnp.where(rows < n0v, cand, xsr[...])

        @pl.when(jnp.logical_not(do))
        def _():
            candr[...] = xsr[...]
            outr[...] = xsr[...]

    w = lambda i, s: (0, 0)
    grid_spec = pltpu.PrefetchScalarGridSpec(
        num_scalar_prefetch=1,
        grid=(NBLK,),
        in_specs=[
            pl.BlockSpec((BR, D), lambda i, s: (i, 0)),
            pl.BlockSpec((NC, BR, D),
                         lambda i, s: (0, jnp.minimum(i, AB - 1), 0)),
            pl.BlockSpec((D, D), w), pl.BlockSpec((1, D), w),
            pl.BlockSpec((D, D), w), pl.BlockSpec((1, D), w),
            pl.BlockSpec((2 * D, D), w), pl.BlockSpec((1, D), w),
        ],
        out_specs=[pl.BlockSpec((BR, D), lambda i, s: (i, 0)),
                   pl.BlockSpec((BR, D), lambda i, s: (i, 0))],
    )
    return pl.pallas_call(
        body,
        grid_spec=grid_spec,
        out_shape=[jax.ShapeDtypeStruct((N, D), jnp.float32),
                   jax.ShapeDtypeStruct((N, D), jnp.float32)],
    )(n0a, xs, acc, W1, b1, W2, b2, cW, cb)


def _combine_pass2(n0a, xs, acc, W1, b1, W2, b2, cW, cb):
    def body(n0s, xsr, ar, W1r, b1r, W2r, b2r, cWr, cbr, outr):
        i = pl.program_id(0)
        rows = i * BR + lax.broadcasted_iota(jnp.int32, (BR, 1), 0)
        n0v = n0s[0]
        do = (i + 1) * BR > n0v

        @pl.when(do)
        def _():
            w0 = i * BR - (n0v // CH) * CH
            chunks = []
            for c in range(BR // CH):
                s = jnp.clip(w0 + c * CH, 0, ACC - CH)
                s = pl.multiple_of(s, 8)
                chunks.append(ar[0, pl.ds(s, CH), :] + ar[1, pl.ds(s, CH), :])
            agg = jnp.concatenate(chunks, axis=0)
            agg = jnp.where((rows >= n0v) & (rows < n0v + 9500), agg, 0.0)
            cand = _mlp_combine(xsr[...], W1r[...], b1r[...], W2r[...],
                                b2r[...], cWr[:D], cWr[D:], cbr[...], agg)
            outr[...] = jnp.where(rows >= n0v, cand, xsr[...])

        @pl.when(jnp.logical_not(do))
        def _():
            outr[...] = xsr[...]

    w = lambda i, s: (0, 0)
    grid_spec = pltpu.PrefetchScalarGridSpec(
        num_scalar_prefetch=1,
        grid=(NBLK,),
        in_specs=[
            pl.BlockSpec((BR, D), lambda i, s: (i, 0)),
            pl.BlockSpec((NC, ACC, D), lambda i, s: (0, 0, 0)),
            pl.BlockSpec((D, D), w), pl.BlockSpec((1, D), w),
            pl.BlockSpec((D, D), w), pl.BlockSpec((1, D), w),
            pl.BlockSpec((2 * D, D), w), pl.BlockSpec((1, D), w),
        ],
        out_specs=pl.BlockSpec((BR, D), lambda i, s: (i, 0)),
    )
    return pl.pallas_call(
        body,
        grid_spec=grid_spec,
        out_shape=jax.ShapeDtypeStruct((N, D), jnp.float32),
    )(n0a, xs, acc, W1, b1, W2, b2, cW, cb)


def kernel(xs, k_batch, bipartites_list,
           c1_W1, c1_b1, c1_W2, c1_b2, c1_cW, c1_cb,
           c2_W1, c2_b1, c2_W2, c2_b2, c2_cW, c2_cb):
    kbp = jnp.concatenate(
        [k_batch.astype(jnp.int32), jnp.zeros((480,), jnp.int32)]
    ).reshape(160, D)
    edges = bipartites_list.astype(jnp.int32).reshape(2 * E // D, D)
    n0o, g1, s1, g2, s2 = _prep_indices(kbp, edges)
    n0a = n0o[0]
    zrows = jnp.zeros((STRIPE, D), jnp.float32)

    acc1 = _sc_segment_sum(xs, g1.reshape(NW, NB, K), s1.reshape(NW, NB, K),
                           zrows)
    xs1, cand0 = _combine_pass1(n0a, xs, acc1,
                                c1_W1, c1_b1.reshape(1, D),
                                c1_W2, c1_b2.reshape(1, D),
                                c1_cW, c1_cb.reshape(1, D))

    acc2 = _sc_segment_sum(cand0, g2.reshape(NW, NB, K),
                           s2.reshape(NW, NB, K), zrows)
    xs2 = _combine_pass2(n0a, xs1, acc2,
                         c2_W1, c2_b1.reshape(1, D),
                         c2_W2, c2_b2.reshape(1, D),
                         c2_cW, c2_cb.reshape(1, D))
    return xs2


# speedup vs baseline: 3.9424x; 1.0045x over previous
import functools

import jax
import jax.numpy as jnp
from jax import lax
from jax.experimental import pallas as pl
from jax.experimental.pallas import tpu as pltpu
from jax.experimental.pallas import tpu_sc as plsc

D = 128
N = 20000
E = 320000
NC = 2
NS = 16
NW = NC * NS
K = 128
NB = 80
EW = NB * K
EPAD = NW * EW
ACC = 9728
JBASE = 9600
STRIPE = ACC // NS
BR = 800
NBLK = N // BR
AB = 12
CH = 80


def _prep_indices(k_batch, edges):
    def body(kbr, edr, n0o, g1o, s1o, g2o, s2o):
        n0 = jnp.int32(N) - jnp.sum(kbr[...])
        n1 = jnp.int32(N) - n0
        e0 = edr[:E]
        e1 = edr[E:]
        junk0 = JBASE + (e0 & 127)
        junk1 = JBASE + (e1 & 127)
        g1 = n0 + jnp.minimum(e1, n1 - 1)
        g1 = jnp.where(g1 < 0, g1 + N, g1)
        s1 = jnp.where(e0 < n0, e0, junk0)
        g2 = jnp.minimum(e0, n0 - 1)
        g2 = jnp.where(g2 < 0, g2 + N, g2)
        r = n0 - (n0 // CH) * CH
        s2 = jnp.where(e1 < n1, e1 + r, junk1)
        flat = lax.broadcasted_iota(jnp.int32, (EPAD - E,), 0)
        g1o[:E] = g1
        s1o[:E] = s1
        g2o[:E] = g2
        s2o[:E] = s2
        g1o[E:] = flat
        s1o[E:] = JBASE + (flat & 127)
        g2o[E:] = flat
        s2o[E:] = JBASE + (flat & 127)
        n0o[0, 0] = n0

    return pl.pallas_call(
        body,
        in_specs=[pl.BlockSpec((N,), lambda: (0,)),
                  pl.BlockSpec((2 * E,), lambda: (0,))],
        out_specs=[pl.BlockSpec(memory_space=pltpu.SMEM),
                   pl.BlockSpec((EPAD,), lambda: (0,)),
                   pl.BlockSpec((EPAD,), lambda: (0,)),
                   pl.BlockSpec((EPAD,), lambda: (0,)),
                   pl.BlockSpec((EPAD,), lambda: (0,))],
        out_shape=[jax.ShapeDtypeStruct((1, 1), jnp.int32)]
        + [jax.ShapeDtypeStruct((EPAD,), jnp.int32)] * 4,
    )(k_batch, edges)


def _sc_segment_sum(table, gidx, sidx, zrows):
    mesh = plsc.VectorSubcoreMesh(
        core_axis_name="c", subcore_axis_name="s",
        num_cores=NC, num_subcores=NS)

    @functools.partial(
        pl.kernel,
        out_type=jax.ShapeDtypeStruct((NC, ACC, D), jnp.float32),
        mesh=mesh,
        scratch_types=[
            pltpu.VMEM((NB, K), jnp.int32),
            pltpu.VMEM((NB, K), jnp.int32),
            pltpu.VMEM((K, D), jnp.float32),
            pltpu.VMEM((K, D), jnp.float32),
            pltpu.VMEM_SHARED((ACC, D), jnp.float32),
            pltpu.SemaphoreType.DMA,
            pltpu.SemaphoreType.DMA,
        ],
    )
    def seg_kernel(table_h, gidx_h, sidx_h, z_h, out_h,
                   gall, sall, rowsA, rowsB, acc, semA, semB):
        cid = lax.axis_index("c")
        sid = lax.axis_index("s")
        wid = sid * NC + cid
        pltpu.sync_copy(gidx_h.at[wid], gall)
        pltpu.sync_copy(sidx_h.at[wid], sall)
        pltpu.async_copy(table_h.at[gall.at[0]], rowsA, semA)
        pltpu.sync_copy(z_h, acc.at[pl.ds(sid * STRIPE, STRIPE)])
        plsc.subcore_barrier()

        def waitA(b):
            pltpu.make_async_copy(table_h.at[gall.at[b]], rowsA, semA).wait()

        def waitB(b):
            pltpu.make_async_copy(table_h.at[gall.at[b]], rowsB, semB).wait()

        def body(t, carry):
            b0 = 2 * t
            b1 = b0 + 1
            pltpu.async_copy(table_h.at[gall.at[b1]], rowsB, semB)
            waitA(b0)
            pltpu.sync_copy(rowsA, acc.at[sall.at[b0]], add=True)
            pltpu.async_copy(table_h.at[gall.at[b0 + 2]], rowsA, semA)
            waitB(b1)
            pltpu.sync_copy(rowsB, acc.at[sall.at[b1]], add=True)
            return carry

        lax.fori_loop(0, NB // 2 - 1, body, 0)
        pltpu.async_copy(table_h.at[gall.at[NB - 1]], rowsB, semB)
        waitA(NB - 2)
        pltpu.sync_copy(rowsA, acc.at[sall.at[NB - 2]], add=True)
        waitB(NB - 1)
        pltpu.sync_copy(rowsB, acc.at[sall.at[NB - 1]], add=True)
        plsc.subcore_barrier()
        pltpu.sync_copy(acc.at[pl.ds(sid * STRIPE, STRIPE)],
                        out_h.at[cid, pl.ds(sid * STRIPE, STRIPE)])

    return seg_kernel(table, gidx, sidx, zrows)


def _mlp_combine(x, W1, b1, W2, b2, cWx, cWh, cb, agg):
    bf = jnp.bfloat16
    h = jnp.maximum(
        jnp.dot(agg.astype(bf), W1.astype(bf),
                preferred_element_type=jnp.float32) + b1, 0.0)
    h = jnp.maximum(
        jnp.dot(h.astype(bf), W2.astype(bf),
                preferred_element_type=jnp.float32) + b2, 0.0)
    cand = jnp.dot(x.astype(bf), cWx.astype(bf),
                   preferred_element_type=jnp.float32)
    cand = cand + jnp.dot(h.astype(bf), cWh.astype(bf),
                          preferred_element_type=jnp.float32) + cb
    return jnp.maximum(cand, 0.0)


def _combine_pass1(n0a, xs, acc, W1, b1, W2, b2, cW, cb):
    def body(n0s, xsr, ar, W1r, b1r, W2r, b2r, cWr, cbr, outr, candr):
        i = pl.program_id(0)
        rows = i * BR + lax.broadcasted_iota(jnp.int32, (BR, 1), 0)
        n0v = n0s[0]
        do = (i * BR < n0v) | (n0v == 0)

        @pl.when(do)
        def _():
            agg = jnp.where(rows < 9500, ar[0] + ar[1], 0.0)
            cand = _mlp_combine(xsr[...], W1r[...], b1r[...], W2r[...],
                                b2r[...], cWr[:D], cWr[D:], cbr[...], agg)
            candr[...] = cand
            outr[...] = jnp.where(rows < n0v, cand, xsr[...])

        @pl.when(jnp.logical_not(do))
        def _():
            candr[...] = xsr[...]
            outr[...] = xsr[...]

    w = lambda i, s: (0, 0)
    grid_spec = pltpu.PrefetchScalarGridSpec(
        num_scalar_prefetch=1,
        grid=(NBLK,),
        in_specs=[
            pl.BlockSpec((BR, D), lambda i, s: (i, 0)),
            pl.BlockSpec((NC, BR, D),
                         lambda i, s: (0, jnp.minimum(i, AB - 1), 0)),
            pl.BlockSpec((D, D), w), pl.BlockSpec((1, D), w),
            pl.BlockSpec((D, D), w), pl.BlockSpec((1, D), w),
            pl.BlockSpec((2 * D, D), w), pl.BlockSpec((1, D), w),
        ],
        out_specs=[pl.BlockSpec((BR, D), lambda i, s: (i, 0)),
                   pl.BlockSpec((BR, D), lambda i, s: (i, 0))],
    )
    return pl.pallas_call(
        body,
        grid_spec=grid_spec,
        out_shape=[jax.ShapeDtypeStruct((N, D), jnp.float32),
                   jax.ShapeDtypeStruct((N, D), jnp.float32)],
    )(n0a, xs, acc, W1, b1, W2, b2, cW, cb)


def _combine_pass2(n0a, xs, acc, W1, b1, W2, b2, cW, cb):
    def body(n0s, xsr, ar, W1r, b1r, W2r, b2r, cWr, cbr, outr):
        i = pl.program_id(0)
        rows = i * BR + lax.broadcasted_iota(jnp.int32, (BR, 1), 0)
        n0v = n0s[0]
        do = (i + 1) * BR > n0v

        @pl.when(do)
        def _():
            w0 = i * BR - (n0v // CH) * CH
            chunks = []
            for c in range(BR // CH):
                s = jnp.clip(w0 + c * CH, 0, ACC - CH)
                s = pl.multiple_of(s, 8)
                chunks.append(ar[0, pl.ds(s, CH), :] + ar[1, pl.ds(s, CH), :])
            agg = jnp.concatenate(chunks, axis=0)
            agg = jnp.where((rows >= n0v) & (rows < n0v + 9500), agg, 0.0)
            cand = _mlp_combine(xsr[...], W1r[...], b1r[...], W2r[...],
                                b2r[...], cWr[:D], cWr[D:], cbr[...], agg)
            outr[...] = jnp.where(rows >= n0v, cand, xsr[...])

        @pl.when(jnp.logical_not(do))
        def _():
            outr[...] = xsr[...]

    w = lambda i, s: (0, 0)
    grid_spec = pltpu.PrefetchScalarGridSpec(
        num_scalar_prefetch=1,
        grid=(NBLK,),
        in_specs=[
            pl.BlockSpec((BR, D), lambda i, s: (i, 0)),
            pl.BlockSpec((NC, ACC, D), lambda i, s: (0, 0, 0)),
            pl.BlockSpec((D, D), w), pl.BlockSpec((1, D), w),
            pl.BlockSpec((D, D), w), pl.BlockSpec((1, D), w),
            pl.BlockSpec((2 * D, D), w), pl.BlockSpec((1, D), w),
        ],
        out_specs=pl.BlockSpec((BR, D), lambda i, s: (i, 0)),
    )
    return pl.pallas_call(
        body,
        grid_spec=grid_spec,
        out_shape=jax.ShapeDtypeStruct((N, D), jnp.float32),
    )(n0a, xs, acc, W1, b1, W2, b2, cW, cb)


def kernel(xs, k_batch, bipartites_list,
           c1_W1, c1_b1, c1_W2, c1_b2, c1_cW, c1_cb,
           c2_W1, c2_b1, c2_W2, c2_b2, c2_cW, c2_cb):
    edges = bipartites_list.astype(jnp.int32).reshape(2 * E)
    n0o, g1, s1, g2, s2 = _prep_indices(k_batch.astype(jnp.int32), edges)
    n0a = n0o[0]
    zrows = jnp.zeros((STRIPE, D), jnp.float32)

    acc1 = _sc_segment_sum(xs, g1.reshape(NW, NB, K), s1.reshape(NW, NB, K),
                           zrows)
    xs1, cand0 = _combine_pass1(n0a, xs, acc1,
                                c1_W1, c1_b1.reshape(1, D),
                                c1_W2, c1_b2.reshape(1, D),
                                c1_cW, c1_cb.reshape(1, D))

    acc2 = _sc_segment_sum(cand0, g2.reshape(NW, NB, K),
                           s2.reshape(NW, NB, K), zrows)
    xs2 = _combine_pass2(n0a, xs1, acc2,
                         c2_W1, c2_b1.reshape(1, D),
                         c2_W2, c2_b2.reshape(1, D),
                         c2_cW, c2_cb.reshape(1, D))
    return xs2


# speedup vs baseline: 3.9967x; 1.0138x over previous
import functools

import jax
import jax.numpy as jnp
from jax import lax
from jax.experimental import pallas as pl
from jax.experimental.pallas import tpu as pltpu
from jax.experimental.pallas import tpu_sc as plsc

D = 128
N = 20000
E = 320000
NC = 2
NS = 16
NW = NC * NS
K = 128
NB = 80
EW = NB * K
EPAD = NW * EW
ACC = 9728
JBASE = 9600
STRIPE = ACC // NS
BR = 800
NBLK = N // BR
AB = 12
CH = 80


def _prep_indices(k_batch, edges):
    def body(kbr, edr, n0o, g1o, s1o, g2o, s2o):
        n0 = jnp.int32(N) - jnp.sum(kbr[...])
        n1 = jnp.int32(N) - n0
        e0 = edr[0, 0]
        e1 = edr[0, 1]
        junk0 = JBASE + (e0 & 127)
        junk1 = JBASE + (e1 & 127)
        g1 = n0 + jnp.minimum(e1, n1 - 1)
        g1 = jnp.where(g1 < 0, g1 + N, g1)
        s1 = jnp.where(e0 < n0, e0, junk0)
        g2 = jnp.minimum(e0, n0 - 1)
        g2 = jnp.where(g2 < 0, g2 + N, g2)
        r = n0 - (n0 // CH) * CH
        s2 = jnp.where(e1 < n1, e1 + r, junk1)
        flat = lax.broadcasted_iota(jnp.int32, (EPAD - E,), 0)
        g1o[:E] = g1
        s1o[:E] = s1
        g2o[:E] = g2
        s2o[:E] = s2
        g1o[E:] = flat
        s1o[E:] = JBASE + (flat & 127)
        g2o[E:] = flat
        s2o[E:] = JBASE + (flat & 127)
        n0o[0, 0] = n0

    return pl.pallas_call(
        body,
        in_specs=[pl.BlockSpec((N,), lambda: (0,)),
                  pl.BlockSpec((1, 2, E), lambda: (0, 0, 0))],
        out_specs=[pl.BlockSpec(memory_space=pltpu.SMEM),
                   pl.BlockSpec((EPAD,), lambda: (0,)),
                   pl.BlockSpec((EPAD,), lambda: (0,)),
                   pl.BlockSpec((EPAD,), lambda: (0,)),
                   pl.BlockSpec((EPAD,), lambda: (0,))],
        out_shape=[jax.ShapeDtypeStruct((1, 1), jnp.int32)]
        + [jax.ShapeDtypeStruct((EPAD,), jnp.int32)] * 4,
    )(k_batch, edges)


def _sc_segment_sum(table, gidx, sidx, zrows):
    mesh = plsc.VectorSubcoreMesh(
        core_axis_name="c", subcore_axis_name="s",
        num_cores=NC, num_subcores=NS)

    @functools.partial(
        pl.kernel,
        out_type=jax.ShapeDtypeStruct((NC, ACC, D), jnp.float32),
        mesh=mesh,
        scratch_types=[
            pltpu.VMEM((NB, K), jnp.int32),
            pltpu.VMEM((NB, K), jnp.int32),
            pltpu.VMEM((K, D), jnp.float32),
            pltpu.VMEM((K, D), jnp.float32),
            pltpu.VMEM_SHARED((ACC, D), jnp.float32),
            pltpu.SemaphoreType.DMA,
            pltpu.SemaphoreType.DMA,
        ],
    )
    def seg_kernel(table_h, gidx_h, sidx_h, z_h, out_h,
                   gall, sall, rowsA, rowsB, acc, semA, semB):
        cid = lax.axis_index("c")
        sid = lax.axis_index("s")
        wid = sid * NC + cid
        pltpu.sync_copy(gidx_h.at[wid], gall)
        pltpu.sync_copy(sidx_h.at[wid], sall)
        pltpu.async_copy(table_h.at[gall.at[0]], rowsA, semA)
        pltpu.sync_copy(z_h, acc.at[pl.ds(sid * STRIPE, STRIPE)])
        plsc.subcore_barrier()

        def waitA(b):
            pltpu.make_async_copy(table_h.at[gall.at[b]], rowsA, semA).wait()

        def waitB(b):
            pltpu.make_async_copy(table_h.at[gall.at[b]], rowsB, semB).wait()

        def body(t, carry):
            b0 = 2 * t
            b1 = b0 + 1
            pltpu.async_copy(table_h.at[gall.at[b1]], rowsB, semB)
            waitA(b0)
            pltpu.sync_copy(rowsA, acc.at[sall.at[b0]], add=True)
            pltpu.async_copy(table_h.at[gall.at[b0 + 2]], rowsA, semA)
            waitB(b1)
            pltpu.sync_copy(rowsB, acc.at[sall.at[b1]], add=True)
            return carry

        lax.fori_loop(0, NB // 2 - 1, body, 0)
        pltpu.async_copy(table_h.at[gall.at[NB - 1]], rowsB, semB)
        waitA(NB - 2)
        pltpu.sync_copy(rowsA, acc.at[sall.at[NB - 2]], add=True)
        waitB(NB - 1)
        pltpu.sync_copy(rowsB, acc.at[sall.at[NB - 1]], add=True)
        plsc.subcore_barrier()
        pltpu.sync_copy(acc.at[pl.ds(sid * STRIPE, STRIPE)],
                        out_h.at[cid, pl.ds(sid * STRIPE, STRIPE)])

    return seg_kernel(table, gidx, sidx, zrows)


def _mlp_combine(x, W1, b1, W2, b2, cWx, cWh, cb, agg):
    bf = jnp.bfloat16
    h = jnp.maximum(
        jnp.dot(agg.astype(bf), W1.astype(bf),
                preferred_element_type=jnp.float32) + b1, 0.0)
    h = jnp.maximum(
        jnp.dot(h.astype(bf), W2.astype(bf),
                preferred_element_type=jnp.float32) + b2, 0.0)
    cand = jnp.dot(x.astype(bf), cWx.astype(bf),
                   preferred_element_type=jnp.float32)
    cand = cand + jnp.dot(h.astype(bf), cWh.astype(bf),
                          preferred_element_type=jnp.float32) + cb
    return jnp.maximum(cand, 0.0)


def _combine_pass1(n0a, xs, acc, W1, b1, W2, b2, cW, cb):
    def body(n0s, xsr, ar, W1r, b1r, W2r, b2r, cWr, cbr, outr, candr):
        i = pl.program_id(0)
        rows = i * BR + lax.broadcasted_iota(jnp.int32, (BR, 1), 0)
        n0v = n0s[0]
        do = (i * BR < n0v) | (n0v == 0)

        @pl.when(do)
        def _():
            agg = jnp.where(rows < 9500, ar[0] + ar[1], 0.0)
            cand = _mlp_combine(xsr[...], W1r[...], b1r[...], W2r[...],
                                b2r[...], cWr[:D], cWr[D:], cbr[...], agg)
            candr[...] = cand
            outr[...] = jnp.where(rows < n0v, cand, xsr[...])

        @pl.when(jnp.logical_not(do))
        def _():
            candr[...] = xsr[...]
            outr[...] = xsr[...]

    w = lambda i, s: (0, 0)
    grid_spec = pltpu.PrefetchScalarGridSpec(
        num_scalar_prefetch=1,
        grid=(NBLK,),
        in_specs=[
            pl.BlockSpec((BR, D), lambda i, s: (i, 0)),
            pl.BlockSpec((NC, BR, D),
                         lambda i, s: (0, jnp.minimum(i, AB - 1), 0)),
            pl.BlockSpec((D, D), w), pl.BlockSpec((1, D), w),
            pl.BlockSpec((D, D), w), pl.BlockSpec((1, D), w),
            pl.BlockSpec((2 * D, D), w), pl.BlockSpec((1, D), w),
        ],
        out_specs=[pl.BlockSpec((BR, D), lambda i, s: (i, 0)),
                   pl.BlockSpec((BR, D), lambda i, s: (i, 0))],
    )
    return pl.pallas_call(
        body,
        grid_spec=grid_spec,
        out_shape=[jax.ShapeDtypeStruct((N, D), jnp.float32),
                   jax.ShapeDtypeStruct((N, D), jnp.float32)],
    )(n0a, xs, acc, W1, b1, W2, b2, cW, cb)


def _combine_pass2(n0a, xs, acc, W1, b1, W2, b2, cW, cb):
    def body(n0s, xsr, ar, W1r, b1r, W2r, b2r, cWr, cbr, outr):
        i = pl.program_id(0)
        rows = i * BR + lax.broadcasted_iota(jnp.int32, (BR, 1), 0)
        n0v = n0s[0]
        do = (i + 1) * BR > n0v

        @pl.when(do)
        def _():
            w0 = i * BR - (n0v // CH) * CH
            chunks = []
            for c in range(BR // CH):
                s = jnp.clip(w0 + c * CH, 0, ACC - CH)
                s = pl.multiple_of(s, 8)
                chunks.append(ar[0, pl.ds(s, CH), :] + ar[1, pl.ds(s, CH), :])
            agg = jnp.concatenate(chunks, axis=0)
            agg = jnp.where((rows >= n0v) & (rows < n0v + 9500), agg, 0.0)
            cand = _mlp_combine(xsr[...], W1r[...], b1r[...], W2r[...],
                                b2r[...], cWr[:D], cWr[D:], cbr[...], agg)
            outr[...] = jnp.where(rows >= n0v, cand, xsr[...])

        @pl.when(jnp.logical_not(do))
        def _():
            outr[...] = xsr[...]

    w = lambda i, s: (0, 0)
    grid_spec = pltpu.PrefetchScalarGridSpec(
        num_scalar_prefetch=1,
        grid=(NBLK,),
        in_specs=[
            pl.BlockSpec((BR, D), lambda i, s: (i, 0)),
            pl.BlockSpec((NC, ACC, D), lambda i, s: (0, 0, 0)),
            pl.BlockSpec((D, D), w), pl.BlockSpec((1, D), w),
            pl.BlockSpec((D, D), w), pl.BlockSpec((1, D), w),
            pl.BlockSpec((2 * D, D), w), pl.BlockSpec((1, D), w),
        ],
        out_specs=pl.BlockSpec((BR, D), lambda i, s: (i, 0)),
    )
    return pl.pallas_call(
        body,
        grid_spec=grid_spec,
        out_shape=jax.ShapeDtypeStruct((N, D), jnp.float32),
    )(n0a, xs, acc, W1, b1, W2, b2, cW, cb)


def kernel(xs, k_batch, bipartites_list,
           c1_W1, c1_b1, c1_W2, c1_b2, c1_cW, c1_cb,
           c2_W1, c2_b1, c2_W2, c2_b2, c2_cW, c2_cb):
    n0o, g1, s1, g2, s2 = _prep_indices(k_batch.astype(jnp.int32),
                                        bipartites_list.astype(jnp.int32))
    n0a = n0o[0]
    zrows = jnp.zeros((STRIPE, D), jnp.float32)

    acc1 = _sc_segment_sum(xs, g1.reshape(NW, NB, K), s1.reshape(NW, NB, K),
                           zrows)
    xs1, cand0 = _combine_pass1(n0a, xs, acc1,
                                c1_W1, c1_b1.reshape(1, D),
                                c1_W2, c1_b2.reshape(1, D),
                                c1_cW, c1_cb.reshape(1, D))

    acc2 = _sc_segment_sum(cand0, g2.reshape(NW, NB, K),
                           s2.reshape(NW, NB, K), zrows)
    xs2 = _combine_pass2(n0a, xs1, acc2,
                         c2_W1, c2_b1.reshape(1, D),
                         c2_W2, c2_b2.reshape(1, D),
                         c2_cW, c2_cb.reshape(1, D))
    return xs2


# speedup vs baseline: 4.0585x; 1.0155x over previous
import functools

import jax
import jax.numpy as jnp
from jax import lax
from jax.experimental import pallas as pl
from jax.experimental.pallas import tpu as pltpu
from jax.experimental.pallas import tpu_sc as plsc

D = 128
N = 20000
E = 320000
NC = 2
NS = 16
NW = NC * NS
K = 128
NB = 80
EW = NB * K
EPAD = NW * EW
ACC = 9728
JBASE = 9600
STRIPE = ACC // NS
BR = 800
NBLK = N // BR
AB = 12
CH = 80


def _prep_indices(k_batch, edges):
    def body(kbr, edr, n0o, g1o, s1o, g2o, s2o):
        n0 = jnp.int32(N) - jnp.sum(kbr[...])
        n1 = jnp.int32(N) - n0
        e0 = edr[0, 0]
        e1 = edr[0, 1]
        junk0 = JBASE + (e0 & 127)
        junk1 = JBASE + (e1 & 127)
        g1 = n0 + jnp.minimum(e1, n1 - 1)
        g1 = jnp.where(g1 < 0, g1 + N, g1)
        s1 = jnp.where(e0 < n0, e0, junk0)
        g2 = jnp.minimum(e0, n0 - 1)
        g2 = jnp.where(g2 < 0, N + BR - 1, g2)
        r = n0 - (n0 // CH) * CH
        s2 = jnp.where(e1 < n1, e1 + r, junk1)
        flat = lax.broadcasted_iota(jnp.int32, (EPAD - E,), 0)
        g1o[:E] = g1
        s1o[:E] = s1
        g2o[:E] = g2
        s2o[:E] = s2
        g1o[E:] = flat
        s1o[E:] = JBASE + (flat & 127)
        g2o[E:] = flat
        s2o[E:] = JBASE + (flat & 127)
        n0o[0, 0] = n0

    return pl.pallas_call(
        body,
        in_specs=[pl.BlockSpec((N,), lambda: (0,)),
                  pl.BlockSpec((1, 2, E), lambda: (0, 0, 0))],
        out_specs=[pl.BlockSpec(memory_space=pltpu.SMEM),
                   pl.BlockSpec((EPAD,), lambda: (0,)),
                   pl.BlockSpec((EPAD,), lambda: (0,)),
                   pl.BlockSpec((EPAD,), lambda: (0,)),
                   pl.BlockSpec((EPAD,), lambda: (0,))],
        out_shape=[jax.ShapeDtypeStruct((1, 1), jnp.int32)]
        + [jax.ShapeDtypeStruct((EPAD,), jnp.int32)] * 4,
    )(k_batch, edges)


def _sc_segment_sum(table, gidx, sidx, zrows):
    mesh = plsc.VectorSubcoreMesh(
        core_axis_name="c", subcore_axis_name="s",
        num_cores=NC, num_subcores=NS)

    @functools.partial(
        pl.kernel,
        out_type=jax.ShapeDtypeStruct((NC, ACC, D), jnp.float32),
        mesh=mesh,
        scratch_types=[
            pltpu.VMEM((NB, K), jnp.int32),
            pltpu.VMEM((NB, K), jnp.int32),
            pltpu.VMEM((K, D), jnp.float32),
            pltpu.VMEM((K, D), jnp.float32),
            pltpu.VMEM_SHARED((ACC, D), jnp.float32),
            pltpu.SemaphoreType.DMA,
            pltpu.SemaphoreType.DMA,
        ],
    )
    def seg_kernel(table_h, gidx_h, sidx_h, z_h, out_h,
                   gall, sall, rowsA, rowsB, acc, semA, semB):
        cid = lax.axis_index("c")
        sid = lax.axis_index("s")
        wid = sid * NC + cid
        pltpu.sync_copy(gidx_h.at[wid], gall)
        pltpu.sync_copy(sidx_h.at[wid], sall)
        pltpu.async_copy(table_h.at[gall.at[0]], rowsA, semA)
        pltpu.sync_copy(z_h, acc.at[pl.ds(sid * STRIPE, STRIPE)])
        plsc.subcore_barrier()

        def waitA(b):
            pltpu.make_async_copy(table_h.at[gall.at[b]], rowsA, semA).wait()

        def waitB(b):
            pltpu.make_async_copy(table_h.at[gall.at[b]], rowsB, semB).wait()

        def body(t, carry):
            b0 = 2 * t
            b1 = b0 + 1
            pltpu.async_copy(table_h.at[gall.at[b1]], rowsB, semB)
            waitA(b0)
            pltpu.sync_copy(rowsA, acc.at[sall.at[b0]], add=True)
            pltpu.async_copy(table_h.at[gall.at[b0 + 2]], rowsA, semA)
            waitB(b1)
            pltpu.sync_copy(rowsB, acc.at[sall.at[b1]], add=True)
            return carry

        lax.fori_loop(0, NB // 2 - 1, body, 0)
        pltpu.async_copy(table_h.at[gall.at[NB - 1]], rowsB, semB)
        waitA(NB - 2)
        pltpu.sync_copy(rowsA, acc.at[sall.at[NB - 2]], add=True)
        waitB(NB - 1)
        pltpu.sync_copy(rowsB, acc.at[sall.at[NB - 1]], add=True)
        plsc.subcore_barrier()
        pltpu.sync_copy(acc.at[pl.ds(sid * STRIPE, STRIPE)],
                        out_h.at[cid, pl.ds(sid * STRIPE, STRIPE)])

    return seg_kernel(table, gidx, sidx, zrows)


def _mlp_combine(x, W1, b1, W2, b2, cWx, cWh, cb, agg):
    bf = jnp.bfloat16
    h = jnp.maximum(
        jnp.dot(agg.astype(bf), W1.astype(bf),
                preferred_element_type=jnp.float32) + b1, 0.0)
    h = jnp.maximum(
        jnp.dot(h.astype(bf), W2.astype(bf),
                preferred_element_type=jnp.float32) + b2, 0.0)
    cand = jnp.dot(x.astype(bf), cWx.astype(bf),
                   preferred_element_type=jnp.float32)
    cand = cand + jnp.dot(h.astype(bf), cWh.astype(bf),
                          preferred_element_type=jnp.float32) + cb
    return jnp.maximum(cand, 0.0)


def _combine_pass1(n0a, xs, acc, W1, b1, W2, b2, cW, cb):
    def body(n0s, xsr, ar, W1r, b1r, W2r, b2r, cWr, cbr, outr):
        i = pl.program_id(0)
        ib = jnp.minimum(i, NBLK - 1)
        rows = ib * BR + lax.broadcasted_iota(jnp.int32, (BR, 1), 0)
        n0v = n0s[0]
        extra = i == NBLK
        do = (i * BR < n0v) | (n0v == 0) | extra

        @pl.when(do)
        def _():
            agg = jnp.where(rows < 9500, ar[0] + ar[1], 0.0)
            cand = _mlp_combine(xsr[...], W1r[...], b1r[...], W2r[...],
                                b2r[...], cWr[:D], cWr[D:], cbr[...], agg)
            outr[...] = jnp.where(extra | (rows < n0v), cand, xsr[...])

        @pl.when(jnp.logical_not(do))
        def _():
            outr[...] = xsr[...]

    w = lambda i, s: (0, 0)
    xmap = lambda i, s: (jnp.minimum(i, NBLK - 1), 0)
    grid_spec = pltpu.PrefetchScalarGridSpec(
        num_scalar_prefetch=1,
        grid=(NBLK + 1,),
        in_specs=[
            pl.BlockSpec((BR, D), xmap),
            pl.BlockSpec((NC, BR, D),
                         lambda i, s: (0, jnp.minimum(i, AB - 1), 0)),
            pl.BlockSpec((D, D), w), pl.BlockSpec((1, D), w),
            pl.BlockSpec((D, D), w), pl.BlockSpec((1, D), w),
            pl.BlockSpec((2 * D, D), w), pl.BlockSpec((1, D), w),
        ],
        out_specs=pl.BlockSpec((BR, D), lambda i, s: (i, 0)),
    )
    return pl.pallas_call(
        body,
        grid_spec=grid_spec,
        out_shape=jax.ShapeDtypeStruct((N + BR, D), jnp.float32),
    )(n0a, xs, acc, W1, b1, W2, b2, cW, cb)


def _combine_pass2(n0a, xs, acc, W1, b1, W2, b2, cW, cb):
    def body(n0s, xsr, ar, W1r, b1r, W2r, b2r, cWr, cbr, outr):
        i = pl.program_id(0)
        rows = i * BR + lax.broadcasted_iota(jnp.int32, (BR, 1), 0)
        n0v = n0s[0]
        do = (i + 1) * BR > n0v

        @pl.when(do)
        def _():
            w0 = i * BR - (n0v // CH) * CH
            chunks = []
            for c in range(BR // CH):
                s = jnp.clip(w0 + c * CH, 0, ACC - CH)
                s = pl.multiple_of(s, 8)
                chunks.append(ar[0, pl.ds(s, CH), :] + ar[1, pl.ds(s, CH), :])
            agg = jnp.concatenate(chunks, axis=0)
            agg = jnp.where((rows >= n0v) & (rows < n0v + 9500), agg, 0.0)
            cand = _mlp_combine(xsr[...], W1r[...], b1r[...], W2r[...],
                                b2r[...], cWr[:D], cWr[D:], cbr[...], agg)
            outr[...] = jnp.where(rows >= n0v, cand, xsr[...])

        @pl.when(jnp.logical_not(do))
        def _():
            outr[...] = xsr[...]

    w = lambda i, s: (0, 0)
    grid_spec = pltpu.PrefetchScalarGridSpec(
        num_scalar_prefetch=1,
        grid=(NBLK,),
        in_specs=[
            pl.BlockSpec((BR, D), lambda i, s: (i, 0)),
            pl.BlockSpec((NC, ACC, D), lambda i, s: (0, 0, 0)),
            pl.BlockSpec((D, D), w), pl.BlockSpec((1, D), w),
            pl.BlockSpec((D, D), w), pl.BlockSpec((1, D), w),
            pl.BlockSpec((2 * D, D), w), pl.BlockSpec((1, D), w),
        ],
        out_specs=pl.BlockSpec((BR, D), lambda i, s: (i, 0)),
    )
    return pl.pallas_call(
        body,
        grid_spec=grid_spec,
        out_shape=jax.ShapeDtypeStruct((N, D), jnp.float32),
    )(n0a, xs, acc, W1, b1, W2, b2, cW, cb)


def kernel(xs, k_batch, bipartites_list,
           c1_W1, c1_b1, c1_W2, c1_b2, c1_cW, c1_cb,
           c2_W1, c2_b1, c2_W2, c2_b2, c2_cW, c2_cb):
    n0o, g1, s1, g2, s2 = _prep_indices(k_batch.astype(jnp.int32),
                                        bipartites_list.astype(jnp.int32))
    n0a = n0o[0]
    zrows = jnp.zeros((STRIPE, D), jnp.float32)

    acc1 = _sc_segment_sum(xs, g1.reshape(NW, NB, K), s1.reshape(NW, NB, K),
                           zrows)
    mix = _combine_pass1(n0a, xs, acc1,
                         c1_W1, c1_b1.reshape(1, D),
                         c1_W2, c1_b2.reshape(1, D),
                         c1_cW, c1_cb.reshape(1, D))

    acc2 = _sc_segment_sum(mix, g2.reshape(NW, NB, K),
                           s2.reshape(NW, NB, K), zrows)
    xs2 = _combine_pass2(n0a, mix, acc2,
                         c2_W1, c2_b1.reshape(1, D),
                         c2_W2, c2_b2.reshape(1, D),
                         c2_cW, c2_cb.reshape(1, D))
    return xs2


# speedup vs baseline: 4.0647x; 1.0015x over previous
import functools

import jax
import jax.numpy as jnp
from jax import lax
from jax.experimental import pallas as pl
from jax.experimental.pallas import tpu as pltpu
from jax.experimental.pallas import tpu_sc as plsc

D = 128
N = 20000
E = 320000
NC = 2
NS = 16
NW = NC * NS
K = 128
NB = 80
EW = NB * K
EPAD = NW * EW
ACC = 9728
JBASE = 9600
STRIPE = ACC // NS
BR = 800
NBLK = N // BR
AB = 12
CH = 80


def _prep_indices(k_batch, edges):
    def body(kbr, edr, n0o, g1o, s1o, g2o, s2o):
        n0 = jnp.int32(N) - jnp.sum(kbr[...])
        n1 = jnp.int32(N) - n0
        e0 = edr[0, 0]
        e1 = edr[0, 1]
        junk0 = JBASE + (e0 & 127)
        junk1 = JBASE + (e1 & 127)
        g1 = n0 + jnp.minimum(e1, n1 - 1)
        g1 = jnp.where(g1 < 0, g1 + N, g1)
        s1 = jnp.where(e0 < n0, e0, junk0)
        g2 = jnp.minimum(e0, n0 - 1)
        g2 = jnp.where(g2 < 0, N + BR - 1, g2)
        r = n0 - (n0 // CH) * CH
        s2 = jnp.where(e1 < n1, e1 + r, junk1)
        flat = lax.broadcasted_iota(jnp.int32, (EPAD - E,), 0)
        g1o[:E] = g1
        s1o[:E] = s1
        g2o[:E] = g2
        s2o[:E] = s2
        g1o[E:] = flat
        s1o[E:] = JBASE + (flat & 127)
        g2o[E:] = flat
        s2o[E:] = JBASE + (flat & 127)
        n0o[0, 0] = n0

    return pl.pallas_call(
        body,
        in_specs=[pl.BlockSpec((N,), lambda: (0,)),
                  pl.BlockSpec((1, 2, E), lambda: (0, 0, 0))],
        out_specs=[pl.BlockSpec(memory_space=pltpu.SMEM),
                   pl.BlockSpec((EPAD,), lambda: (0,)),
                   pl.BlockSpec((EPAD,), lambda: (0,)),
                   pl.BlockSpec((EPAD,), lambda: (0,)),
                   pl.BlockSpec((EPAD,), lambda: (0,))],
        out_shape=[jax.ShapeDtypeStruct((1, 1), jnp.int32)]
        + [jax.ShapeDtypeStruct((EPAD,), jnp.int32)] * 4,
    )(k_batch, edges)


def _sc_segment_sum(table, gidx, sidx, zrows):
    mesh = plsc.VectorSubcoreMesh(
        core_axis_name="c", subcore_axis_name="s",
        num_cores=NC, num_subcores=NS)

    @functools.partial(
        pl.kernel,
        out_type=jax.ShapeDtypeStruct((NC, ACC, D), jnp.float32),
        mesh=mesh,
        scratch_types=[
            pltpu.VMEM((NB, K), jnp.int32),
            pltpu.VMEM((NB, K), jnp.int32),
            pltpu.VMEM((K, D), jnp.float32),
            pltpu.VMEM((K, D), jnp.float32),
            pltpu.VMEM_SHARED((ACC, D), jnp.float32),
            pltpu.SemaphoreType.DMA,
            pltpu.SemaphoreType.DMA,
        ],
    )
    def seg_kernel(table_h, gidx_h, sidx_h, z_h, out_h,
                   gall, sall, rowsA, rowsB, acc, semA, semB):
        cid = lax.axis_index("c")
        sid = lax.axis_index("s")
        wid = sid * NC + cid
        pltpu.async_copy(gidx_h.at[wid], gall, semA)
        pltpu.async_copy(sidx_h.at[wid], sall, semB)
        pltpu.make_async_copy(gidx_h.at[wid], gall, semA).wait()
        pltpu.async_copy(table_h.at[gall.at[0]], rowsA, semA)
        pltpu.sync_copy(z_h, acc.at[pl.ds(sid * STRIPE, STRIPE)])
        pltpu.make_async_copy(sidx_h.at[wid], sall, semB).wait()
        plsc.subcore_barrier()

        def waitA(b):
            pltpu.make_async_copy(table_h.at[gall.at[b]], rowsA, semA).wait()

        def waitB(b):
            pltpu.make_async_copy(table_h.at[gall.at[b]], rowsB, semB).wait()

        def body(t, carry):
            b0 = 2 * t
            b1 = b0 + 1
            pltpu.async_copy(table_h.at[gall.at[b1]], rowsB, semB)
            waitA(b0)
            pltpu.sync_copy(rowsA, acc.at[sall.at[b0]], add=True)
            pltpu.async_copy(table_h.at[gall.at[b0 + 2]], rowsA, semA)
            waitB(b1)
            pltpu.sync_copy(rowsB, acc.at[sall.at[b1]], add=True)
            return carry

        lax.fori_loop(0, NB // 2 - 1, body, 0)
        pltpu.async_copy(table_h.at[gall.at[NB - 1]], rowsB, semB)
        waitA(NB - 2)
        pltpu.sync_copy(rowsA, acc.at[sall.at[NB - 2]], add=True)
        waitB(NB - 1)
        pltpu.sync_copy(rowsB, acc.at[sall.at[NB - 1]], add=True)
        plsc.subcore_barrier()
        pltpu.sync_copy(acc.at[pl.ds(sid * STRIPE, STRIPE)],
                        out_h.at[cid, pl.ds(sid * STRIPE, STRIPE)])

    return seg_kernel(table, gidx, sidx, zrows)


def _mlp_combine(x, W1, b1, W2, b2, cWx, cWh, cb, agg):
    bf = jnp.bfloat16
    h = jnp.maximum(
        jnp.dot(agg.astype(bf), W1, preferred_element_type=jnp.float32)
        + b1, 0.0)
    h = jnp.maximum(
        jnp.dot(h.astype(bf), W2, preferred_element_type=jnp.float32)
        + b2, 0.0)
    cand = jnp.dot(x.astype(bf), cWx, preferred_element_type=jnp.float32)
    cand = cand + jnp.dot(h.astype(bf), cWh,
                          preferred_element_type=jnp.float32) + cb
    return jnp.maximum(cand, 0.0)


def _combine_pass1(n0a, xs, acc, W1, b1, W2, b2, cW, cb):
    def body(n0s, xsr, ar, W1r, b1r, W2r, b2r, cWr, cbr, outr):
        i = pl.program_id(0)
        ib = jnp.minimum(i, NBLK - 1)
        rows = ib * BR + lax.broadcasted_iota(jnp.int32, (BR, 1), 0)
        n0v = n0s[0]
        extra = i == NBLK
        do = (i * BR < n0v) | (n0v == 0) | extra

        @pl.when(do)
        def _():
            agg = jnp.where(rows < 9500, ar[0] + ar[1], 0.0)
            cand = _mlp_combine(xsr[...], W1r[...], b1r[...], W2r[...],
                                b2r[...], cWr[:D], cWr[D:], cbr[...], agg)
            outr[...] = jnp.where(extra | (rows < n0v), cand, xsr[...])

        @pl.when(jnp.logical_not(do))
        def _():
            outr[...] = xsr[...]

    w = lambda i, s: (0, 0)
    xmap = lambda i, s: (jnp.minimum(i, NBLK - 1), 0)
    grid_spec = pltpu.PrefetchScalarGridSpec(
        num_scalar_prefetch=1,
        grid=(NBLK + 1,),
        in_specs=[
            pl.BlockSpec((BR, D), xmap),
            pl.BlockSpec((NC, BR, D),
                         lambda i, s: (0, jnp.minimum(i, AB - 1), 0)),
            pl.BlockSpec((D, D), w), pl.BlockSpec((1, D), w),
            pl.BlockSpec((D, D), w), pl.BlockSpec((1, D), w),
            pl.BlockSpec((2 * D, D), w), pl.BlockSpec((1, D), w),
        ],
        out_specs=pl.BlockSpec((BR, D), lambda i, s: (i, 0)),
    )
    return pl.pallas_call(
        body,
        grid_spec=grid_spec,
        out_shape=jax.ShapeDtypeStruct((N + BR, D), jnp.float32),
    )(n0a, xs, acc, W1, b1, W2, b2, cW, cb)


def _combine_pass2(n0a, xs, acc, W1, b1, W2, b2, cW, cb):
    def body(n0s, xsr, ar, W1r, b1r, W2r, b2r, cWr, cbr, outr):
        i = pl.program_id(0)
        rows = i * BR + lax.broadcasted_iota(jnp.int32, (BR, 1), 0)
        n0v = n0s[0]
        do = (i + 1) * BR > n0v

        @pl.when(do)
        def _():
            w0 = i * BR - (n0v // CH) * CH
            chunks = []
            for c in range(BR // CH):
                s = jnp.clip(w0 + c * CH, 0, ACC - CH)
                s = pl.multiple_of(s, 8)
                chunks.append(ar[0, pl.ds(s, CH), :] + ar[1, pl.ds(s, CH), :])
            agg = jnp.concatenate(chunks, axis=0)
            agg = jnp.where((rows >= n0v) & (rows < n0v + 9500), agg, 0.0)
            cand = _mlp_combine(xsr[...], W1r[...], b1r[...], W2r[...],
                                b2r[...], cWr[:D], cWr[D:], cbr[...], agg)
            outr[...] = jnp.where(rows >= n0v, cand, xsr[...])

        @pl.when(jnp.logical_not(do))
        def _():
            outr[...] = xsr[...]

    w = lambda i, s: (0, 0)
    grid_spec = pltpu.PrefetchScalarGridSpec(
        num_scalar_prefetch=1,
        grid=(NBLK,),
        in_specs=[
            pl.BlockSpec((BR, D), lambda i, s: (i, 0)),
            pl.BlockSpec((NC, ACC, D), lambda i, s: (0, 0, 0)),
            pl.BlockSpec((D, D), w), pl.BlockSpec((1, D), w),
            pl.BlockSpec((D, D), w), pl.BlockSpec((1, D), w),
            pl.BlockSpec((2 * D, D), w), pl.BlockSpec((1, D), w),
        ],
        out_specs=pl.BlockSpec((BR, D), lambda i, s: (i, 0)),
    )
    return pl.pallas_call(
        body,
        grid_spec=grid_spec,
        out_shape=jax.ShapeDtypeStruct((N, D), jnp.float32),
    )(n0a, xs, acc, W1, b1, W2, b2, cW, cb)


def kernel(xs, k_batch, bipartites_list,
           c1_W1, c1_b1, c1_W2, c1_b2, c1_cW, c1_cb,
           c2_W1, c2_b1, c2_W2, c2_b2, c2_cW, c2_cb):
    n0o, g1, s1, g2, s2 = _prep_indices(k_batch.astype(jnp.int32),
                                        bipartites_list.astype(jnp.int32))
    n0a = n0o[0]
    zrows = jnp.zeros((STRIPE, D), jnp.float32)

    acc1 = _sc_segment_sum(xs, g1.reshape(NW, NB, K), s1.reshape(NW, NB, K),
                           zrows)
    bf = jnp.bfloat16
    mix = _combine_pass1(n0a, xs, acc1,
                         c1_W1.astype(bf), c1_b1.reshape(1, D),
                         c1_W2.astype(bf), c1_b2.reshape(1, D),
                         c1_cW.astype(bf), c1_cb.reshape(1, D))

    acc2 = _sc_segment_sum(mix, g2.reshape(NW, NB, K),
                           s2.reshape(NW, NB, K), zrows)
    xs2 = _combine_pass2(n0a, mix, acc2,
                         c2_W1.astype(bf), c2_b1.reshape(1, D),
                         c2_W2.astype(bf), c2_b2.reshape(1, D),
                         c2_cW.astype(bf), c2_cb.reshape(1, D))
    return xs2
